# Initial kernel scaffold; baseline (speedup 1.0000x reference)
import functools

import jax
import jax.numpy as jnp
from jax import lax
from jax.experimental import pallas as pl
from jax.experimental.pallas import tpu as pltpu
from jax.experimental.pallas import tpu_sc as plsc

NU = 100000
NI = 100000
NE = 3200000
NC, NS, L = 2, 16, 16
NW = NC * NS
NR = NE // L
RPT = NR // NW
CHUNK = 125
NCHUNK = RPT // CHUNK

_MESH = plsc.VectorSubcoreMesh(core_axis_name="c", subcore_axis_name="s")

_f32 = jnp.float32
_i32 = jnp.int32


def _wid():
    return lax.axis_index("s") * NC + lax.axis_index("c")


@functools.partial(
    pl.kernel,
    out_type=jax.ShapeDtypeStruct((NC, NI), _f32),
    mesh=_MESH,
    scratch_types=[
        pltpu.VMEM((CHUNK, L), _i32),
        pltpu.VMEM((CHUNK, L), _f32),
        pltpu.VMEM_SHARED((NI,), _f32),
    ],
)
def _pass_counts(c2, ones2, zeros, out, idx_v, ones_v, acc):
    c = lax.axis_index("c")
    s = lax.axis_index("s")
    rowbase = _wid() * RPT

    @pl.when(s == 0)
    def _():
        pltpu.sync_copy(zeros, acc)

    pltpu.sync_copy(ones2, ones_v)
    plsc.subcore_barrier()

    def body(k, carry):
        r0 = rowbase + k * CHUNK
        pltpu.sync_copy(c2.at[pl.ds(r0, CHUNK)], idx_v)
        pltpu.sync_copy(ones_v, acc.at[idx_v], add=True)
        return carry

    lax.fori_loop(0, NCHUNK, body, 0)
    plsc.subcore_barrier()

    @pl.when(s == 0)
    def _():
        pltpu.sync_copy(acc, out.at[c])


@functools.partial(
    pl.kernel,
    out_type=jax.ShapeDtypeStruct((NC, NI), _f32),
    mesh=_MESH,
    scratch_types=[
        pltpu.VMEM((NI,), _f32),
        pltpu.VMEM((CHUNK, L), _i32),
        pltpu.VMEM((CHUNK, L), _f32),
        pltpu.VMEM((CHUNK, L), _f32),
        pltpu.VMEM_SHARED((NI,), _f32),
    ],
)
def _pass_pop(c2, v2, lc, zeros, out, lc_v, idx_v, val_v, res_v, acc):
    c = lax.axis_index("c")
    s = lax.axis_index("s")
    rowbase = _wid() * RPT

    @pl.when(s == 0)
    def _():
        pltpu.sync_copy(zeros, acc)

    pltpu.sync_copy(lc, lc_v)
    plsc.subcore_barrier()

    def body(k, carry):
        r0 = rowbase + k * CHUNK
        pltpu.sync_copy(c2.at[pl.ds(r0, CHUNK)], idx_v)
        pltpu.sync_copy(v2.at[pl.ds(r0, CHUNK)], val_v)
        for r in range(CHUNK):
            g = plsc.load_gather(lc_v, [idx_v[r]])
            res_v[r] = val_v[r] * g
        pltpu.sync_copy(res_v, acc.at[idx_v], add=True)
        return carry

    lax.fori_loop(0, NCHUNK, body, 0)
    plsc.subcore_barrier()

    @pl.when(s == 0)
    def _():
        pltpu.sync_copy(acc, out.at[c])


@functools.partial(
    pl.kernel,
    out_type=[
        jax.ShapeDtypeStruct((NC, NU), _f32),
        jax.ShapeDtypeStruct((NC, NI), _f32),
        jax.ShapeDtypeStruct((NC, NI), _f32),
    ],
    mesh=_MESH,
    scratch_types=[
        pltpu.VMEM((NI,), _i32),
        pltpu.VMEM((CHUNK, L), _i32),
        pltpu.VMEM((CHUNK, L), _i32),
        pltpu.VMEM((CHUNK, L), _f32),
        pltpu.VMEM((CHUNK, L), _f32),
        pltpu.VMEM((CHUNK, L), _f32),
        pltpu.VMEM((CHUNK, L), _f32),
        pltpu.VMEM_SHARED((NU,), _f32),
        pltpu.VMEM_SHARED((NI,), _f32),
        pltpu.VMEM_SHARED((NI,), _f32),
    ],
)
def _pass_act_item(c2, r2, v2, gp, zeros, out_act, out_il, out_ih,
                   gp_v, cidx_v, ridx_v, val_v, res_e, res_t, res_t2,
                   acc_act, acc_il, acc_ih):
    c = lax.axis_index("c")
    s = lax.axis_index("s")
    rowbase = _wid() * RPT

    @pl.when(s == 0)
    def _():
        pltpu.sync_copy(zeros, acc_act)
        pltpu.sync_copy(zeros, acc_il)
        pltpu.sync_copy(zeros, acc_ih)

    pltpu.sync_copy(gp, gp_v)
    plsc.subcore_barrier()

    def body(k, carry):
        r0 = rowbase + k * CHUNK
        pltpu.sync_copy(c2.at[pl.ds(r0, CHUNK)], cidx_v)
        pltpu.sync_copy(r2.at[pl.ds(r0, CHUNK)], ridx_v)
        pltpu.sync_copy(v2.at[pl.ds(r0, CHUNK)], val_v)
        for r in range(CHUNK):
            q = plsc.load_gather(gp_v, [cidx_v[r]])
            g = plsc.bitcast(q << 16, _f32)
            p = plsc.bitcast(q & jnp.int32(-65536), _f32)
            v = val_v[r]
            res_e[r] = v * g
            t = v * p
            res_t[r] = t
            res_t2[r] = t * t
        pltpu.sync_copy(res_e, acc_act.at[ridx_v], add=True)
        pltpu.sync_copy(res_t, acc_il.at[cidx_v], add=True)
        pltpu.sync_copy(res_t2, acc_ih.at[cidx_v], add=True)
        return carry

    lax.fori_loop(0, NCHUNK, body, 0)
    plsc.subcore_barrier()

    @pl.when(s == 0)
    def _():
        pltpu.sync_copy(acc_act, out_act.at[c])
        pltpu.sync_copy(acc_il, out_il.at[c])
        pltpu.sync_copy(acc_ih, out_ih.at[c])


@functools.partial(
    pl.kernel,
    out_type=[
        jax.ShapeDtypeStruct((NC, NU), _f32),
        jax.ShapeDtypeStruct((NC, NU), _f32),
    ],
    mesh=_MESH,
    scratch_types=[
        pltpu.VMEM((NU,), _f32),
        pltpu.VMEM((CHUNK, L), _i32),
        pltpu.VMEM((CHUNK, L), _f32),
        pltpu.VMEM((CHUNK, L), _f32),
        pltpu.VMEM((CHUNK, L), _f32),
        pltpu.VMEM_SHARED((NU,), _f32),
        pltpu.VMEM_SHARED((NU,), _f32),
    ],
)
def _pass_user(r2, v2, a, zeros, out_ul, out_uh,
               a_v, ridx_v, val_v, res_s, res_s2, acc_ul, acc_uh):
    c = lax.axis_index("c")
    s = lax.axis_index("s")
    rowbase = _wid() * RPT

    @pl.when(s == 0)
    def _():
        pltpu.sync_copy(zeros, acc_ul)
        pltpu.sync_copy(zeros, acc_uh)

    pltpu.sync_copy(a, a_v)
    plsc.subcore_barrier()

    def body(k, carry):
        r0 = rowbase + k * CHUNK
        pltpu.sync_copy(r2.at[pl.ds(r0, CHUNK)], ridx_v)
        pltpu.sync_copy(v2.at[pl.ds(r0, CHUNK)], val_v)
        for r in range(CHUNK):
            av = plsc.load_gather(a_v, [ridx_v[r]])
            sg = val_v[r] * av
            res_s[r] = sg
            res_s2[r] = sg * sg
        pltpu.sync_copy(res_s, acc_ul.at[ridx_v], add=True)
        pltpu.sync_copy(res_s2, acc_uh.at[ridx_v], add=True)
        return carry

    lax.fori_loop(0, NCHUNK, body, 0)
    plsc.subcore_barrier()

    @pl.when(s == 0)
    def _():
        pltpu.sync_copy(acc_ul, out_ul.at[c])
        pltpu.sync_copy(acc_uh, out_uh.at[c])


_HALF_U = NU // 2
_HALF_I = NI // 2


@functools.partial(
    pl.kernel,
    out_type=jax.ShapeDtypeStruct((4, NR, L), _f32),
    mesh=_MESH,
    scratch_types=[
        pltpu.VMEM((_HALF_U,), _i32),
        pltpu.VMEM((_HALF_I,), _i32),
        pltpu.VMEM((CHUNK, L), _i32),
        pltpu.VMEM((CHUNK, L), _i32),
        pltpu.VMEM((CHUNK, L), _f32),
        pltpu.VMEM((CHUNK, L), _f32),
        pltpu.VMEM((CHUNK, L), _f32),
        pltpu.VMEM((CHUNK, L), _f32),
        pltpu.VMEM((CHUNK, L), _f32),
    ],
)
def _pass_out(r2, c2, v2, uwq, iwq, out,
              uwq_v, iwq_v, ridx_v, cidx_v, val_v, o1_v, o2_v, o3_v, o4_v):
    rowbase = _wid() * RPT

    pltpu.sync_copy(uwq, uwq_v)
    pltpu.sync_copy(iwq, iwq_v)

    def unpack(table, idx, half):
        lo = idx < half
        word = plsc.load_gather(table, [jnp.where(lo, idx, idx - half)])
        bits = jnp.where(lo, word << 16, word & jnp.int32(-65536))
        return plsc.bitcast(bits, _f32)

    def body(k, carry):
        r0 = rowbase + k * CHUNK
        pltpu.sync_copy(r2.at[pl.ds(r0, CHUNK)], ridx_v)
        pltpu.sync_copy(c2.at[pl.ds(r0, CHUNK)], cidx_v)
        pltpu.sync_copy(v2.at[pl.ds(r0, CHUNK)], val_v)
        for r in range(CHUNK):
            uw = unpack(uwq_v, ridx_v[r], _HALF_U)
            iw = unpack(iwq_v, cidx_v[r], _HALF_I)
            v = val_v[r]
            av = v * uw
            bv = v - av
            o1 = av * iw
            o4 = bv * iw
            o1_v[r] = o1
            o2_v[r] = bv - o4
            o3_v[r] = av - o1
            o4_v[r] = o4
        pltpu.sync_copy(o1_v, out.at[0, pl.ds(r0, CHUNK)])
        pltpu.sync_copy(o2_v, out.at[1, pl.ds(r0, CHUNK)])
        pltpu.sync_copy(o3_v, out.at[2, pl.ds(r0, CHUNK)])
        pltpu.sync_copy(o4_v, out.at[3, pl.ds(r0, CHUNK)])
        return carry

    lax.fori_loop(0, NCHUNK, body, 0)


def _rn_bits(x):
    return lax.bitcast_convert_type(x, _i32) + jnp.int32(0x8000)


def _tc_log_counts(counts_p):
    def body(cp, lc):
        lc[...] = jnp.log1p(cp[0:8] + cp[8:16])

    return pl.pallas_call(
        body, out_shape=jax.ShapeDtypeStruct((8, 12500), _f32))(counts_p)


def _tc_pop_pack(pop_p):
    def body(pp, gp):
        pr = pp[0:8] + pp[8:16]
        nrm = jnp.sqrt(jnp.sum(pr * pr))
        pop = pr / (nrm + 1e-8)
        g = 1.0 / jnp.log1p(pop + 1e-8)
        p = 1.0 + pop
        gp[...] = (_rn_bits(p) & jnp.int32(-65536)) | (
            (_rn_bits(g) >> 16) & jnp.int32(0xFFFF))

    return pl.pallas_call(
        body, out_shape=jax.ShapeDtypeStruct((8, 12500), _i32))(pop_p)


def _tc_act(act_p):
    def body(ap, a):
        ar = ap[0:8] + ap[8:16]
        nrm = jnp.sqrt(jnp.sum(ar * ar))
        a[...] = 1.0 + ar / (nrm + 1e-8)

    return pl.pallas_call(
        body, out_shape=jax.ShapeDtypeStruct((8, 12500), _f32))(act_p)


def _tc_weights(ul_p, uh_p, il_p, ih_p):
    def pack_halves(w):
        lo = w[0:4]
        hi = w[4:8]
        return (_rn_bits(hi) & jnp.int32(-65536)) | (
            (_rn_bits(lo) >> 16) & jnp.int32(0xFFFF))

    def weight(low_p, high_p, denom):
        low = (low_p[0:8] + low_p[8:16]) * (1.0 / denom)
        high = jnp.sqrt(high_p[0:8] + high_p[8:16] + 1e-12)
        al = jnp.maximum(low, 1e-6)
        ah = jnp.maximum(high, 1e-6)
        return al / (al + ah)

    def body(ulp, uhp, ilp, ihp, uwq, iwq):
        uwq[...] = pack_halves(weight(ulp, uhp, float(NI)))
        iwq[...] = pack_halves(weight(ilp, ihp, float(NU)))

    return pl.pallas_call(
        body,
        out_shape=[
            jax.ShapeDtypeStruct((4, 12500), _i32),
            jax.ShapeDtypeStruct((4, 12500), _i32),
        ])(ul_p, uh_p, il_p, ih_p)


def kernel(values, row_idx, col_idx):
    f = values.astype(_f32)
    v2 = f.reshape(NR, L)
    r2 = row_idx.reshape(NR, L)
    c2 = col_idx.reshape(NR, L)
    zeros = jnp.zeros((NI,), _f32)
    ones2 = jnp.ones((CHUNK, L), _f32)

    counts_p = _pass_counts(c2, ones2, zeros)
    lc = _tc_log_counts(counts_p.reshape(16, 12500))

    pop_p = _pass_pop(c2, v2, lc.reshape(NI), zeros)
    gp = _tc_pop_pack(pop_p.reshape(16, 12500))

    act_p, il_p, ih_p = _pass_act_item(c2, r2, v2, gp.reshape(NI), zeros)
    a = _tc_act(act_p.reshape(16, 12500))

    ul_p, uh_p = _pass_user(r2, v2, a.reshape(NU), zeros)
    uwq, iwq = _tc_weights(
        ul_p.reshape(16, 12500), uh_p.reshape(16, 12500),
        il_p.reshape(16, 12500), ih_p.reshape(16, 12500))

    out = _pass_out(r2, c2, v2, uwq.reshape(_HALF_U), iwq.reshape(_HALF_I))
    return out.reshape(4, NE)


# speedup vs baseline: 108.4538x; 108.4538x over previous
import functools

import jax
import jax.numpy as jnp
from jax import lax
from jax.experimental import pallas as pl
from jax.experimental.pallas import tpu as pltpu
from jax.experimental.pallas import tpu_sc as plsc

NU = 100000
NI = 100000
NE = 3200000
NC, NS, L = 2, 16, 16
NW = NC * NS
EPT = NE // NW
CE = 2000
NCHUNK = EPT // CE
NVEC = CE // L
CE_C = 800
NCHUNK_C = EPT // CE_C
NVEC_C = CE_C // L

_MESH = plsc.VectorSubcoreMesh(core_axis_name="c", subcore_axis_name="s")
_SC_PARAMS = pltpu.CompilerParams(needs_layout_passes=False, use_tc_tiling_on_sc=False)

_f32 = jnp.float32
_i32 = jnp.int32


def _wid():
    return lax.axis_index("s") * NC + lax.axis_index("c")


@functools.partial(
    pl.kernel,
    out_type=jax.ShapeDtypeStruct((NC, NI), _f32),
    mesh=_MESH,
    compiler_params=_SC_PARAMS,
    scratch_types=[
        pltpu.VMEM((CE,), _i32),
        pltpu.VMEM((CE,), _f32),
        pltpu.VMEM_SHARED((NI,), _f32),
    ],
)
def _pass_counts(c1, ones1, zeros, out, idx_v, ones_v, acc):
    c = lax.axis_index("c")
    s = lax.axis_index("s")
    base = _wid() * EPT

    @pl.when(s == 0)
    def _():
        pltpu.sync_copy(zeros, acc)

    pltpu.sync_copy(ones1, ones_v)
    plsc.subcore_barrier()

    def body(k, carry):
        e0 = base + k * CE
        pltpu.sync_copy(c1.at[pl.ds(e0, CE)], idx_v)
        pltpu.sync_copy(ones_v, acc.at[idx_v], add=True)
        return carry

    lax.fori_loop(0, NCHUNK, body, 0)
    plsc.subcore_barrier()

    @pl.when(s == 0)
    def _():
        pltpu.sync_copy(acc, out.at[c])


@functools.partial(
    pl.kernel,
    out_type=jax.ShapeDtypeStruct((NC, NI), _f32),
    mesh=_MESH,
    compiler_params=_SC_PARAMS,
    scratch_types=[
        pltpu.VMEM((NI,), _f32),
        pltpu.VMEM((CE,), _i32),
        pltpu.VMEM((CE,), _f32),
        pltpu.VMEM((CE,), _f32),
        pltpu.VMEM_SHARED((NI,), _f32),
    ],
)
def _pass_pop(c1, v1, lc, zeros, out, lc_v, idx_v, val_v, res_v, acc):
    c = lax.axis_index("c")
    s = lax.axis_index("s")
    base = _wid() * EPT

    @pl.when(s == 0)
    def _():
        pltpu.sync_copy(zeros, acc)

    pltpu.sync_copy(lc, lc_v)
    plsc.subcore_barrier()

    def body(k, carry):
        e0 = base + k * CE
        pltpu.sync_copy(c1.at[pl.ds(e0, CE)], idx_v)
        pltpu.sync_copy(v1.at[pl.ds(e0, CE)], val_v)
        for r in range(NVEC):
            sl = pl.ds(r * L, L)
            g = plsc.load_gather(lc_v, [idx_v[sl]])
            res_v[sl] = val_v[sl] * g
        pltpu.sync_copy(res_v, acc.at[idx_v], add=True)
        return carry

    lax.fori_loop(0, NCHUNK, body, 0)
    plsc.subcore_barrier()

    @pl.when(s == 0)
    def _():
        pltpu.sync_copy(acc, out.at[c])


@functools.partial(
    pl.kernel,
    out_type=[
        jax.ShapeDtypeStruct((NC, NU), _f32),
        jax.ShapeDtypeStruct((NC, NI), _f32),
        jax.ShapeDtypeStruct((NC, NI), _f32),
    ],
    mesh=_MESH,
    compiler_params=_SC_PARAMS,
    scratch_types=[
        pltpu.VMEM((NI,), _i32),
        pltpu.VMEM((CE_C,), _i32),
        pltpu.VMEM((CE_C,), _i32),
        pltpu.VMEM((CE_C,), _f32),
        pltpu.VMEM((CE_C,), _f32),
        pltpu.VMEM((CE_C,), _f32),
        pltpu.VMEM((CE_C,), _f32),
        pltpu.VMEM_SHARED((NU,), _f32),
        pltpu.VMEM_SHARED((NI,), _f32),
        pltpu.VMEM_SHARED((NI,), _f32),
    ],
)
def _pass_act_item(c1, r1, v1, gp, zeros, out_act, out_il, out_ih,
                   gp_v, cidx_v, ridx_v, val_v, res_e, res_t, res_t2,
                   acc_act, acc_il, acc_ih):
    c = lax.axis_index("c")
    s = lax.axis_index("s")
    base = _wid() * EPT

    @pl.when(s == 0)
    def _():
        pltpu.sync_copy(zeros, acc_act)
        pltpu.sync_copy(zeros, acc_il)
        pltpu.sync_copy(zeros, acc_ih)

    pltpu.sync_copy(gp, gp_v)
    plsc.subcore_barrier()

    def body(k, carry):
        e0 = base + k * CE_C
        pltpu.sync_copy(c1.at[pl.ds(e0, CE_C)], cidx_v)
        pltpu.sync_copy(r1.at[pl.ds(e0, CE_C)], ridx_v)
        pltpu.sync_copy(v1.at[pl.ds(e0, CE_C)], val_v)
        for r in range(NVEC_C):
            sl = pl.ds(r * L, L)
            q = plsc.load_gather(gp_v, [cidx_v[sl]])
            g = plsc.bitcast(q << 16, _f32)
            p = plsc.bitcast(q & jnp.int32(-65536), _f32)
            v = val_v[sl]
            res_e[sl] = v * g
            t = v * p
            res_t[sl] = t
            res_t2[sl] = t * t
        pltpu.sync_copy(res_e, acc_act.at[ridx_v], add=True)
        pltpu.sync_copy(res_t, acc_il.at[cidx_v], add=True)
        pltpu.sync_copy(res_t2, acc_ih.at[cidx_v], add=True)
        return carry

    lax.fori_loop(0, NCHUNK_C, body, 0)
    plsc.subcore_barrier()

    @pl.when(s == 0)
    def _():
        pltpu.sync_copy(acc_act, out_act.at[c])
        pltpu.sync_copy(acc_il, out_il.at[c])
        pltpu.sync_copy(acc_ih, out_ih.at[c])


@functools.partial(
    pl.kernel,
    out_type=[
        jax.ShapeDtypeStruct((NC, NU), _f32),
        jax.ShapeDtypeStruct((NC, NU), _f32),
    ],
    mesh=_MESH,
    compiler_params=_SC_PARAMS,
    scratch_types=[
        pltpu.VMEM((NU,), _f32),
        pltpu.VMEM((CE,), _i32),
        pltpu.VMEM((CE,), _f32),
        pltpu.VMEM((CE,), _f32),
        pltpu.VMEM((CE,), _f32),
        pltpu.VMEM_SHARED((NU,), _f32),
        pltpu.VMEM_SHARED((NU,), _f32),
    ],
)
def _pass_user(r1, v1, a, zeros, out_ul, out_uh,
               a_v, ridx_v, val_v, res_s, res_s2, acc_ul, acc_uh):
    c = lax.axis_index("c")
    s = lax.axis_index("s")
    base = _wid() * EPT

    @pl.when(s == 0)
    def _():
        pltpu.sync_copy(zeros, acc_ul)
        pltpu.sync_copy(zeros, acc_uh)

    pltpu.sync_copy(a, a_v)
    plsc.subcore_barrier()

    def body(k, carry):
        e0 = base + k * CE
        pltpu.sync_copy(r1.at[pl.ds(e0, CE)], ridx_v)
        pltpu.sync_copy(v1.at[pl.ds(e0, CE)], val_v)
        for r in range(NVEC):
            sl = pl.ds(r * L, L)
            av = plsc.load_gather(a_v, [ridx_v[sl]])
            sg = val_v[sl] * av
            res_s[sl] = sg
            res_s2[sl] = sg * sg
        pltpu.sync_copy(res_s, acc_ul.at[ridx_v], add=True)
        pltpu.sync_copy(res_s2, acc_uh.at[ridx_v], add=True)
        return carry

    lax.fori_loop(0, NCHUNK, body, 0)
    plsc.subcore_barrier()

    @pl.when(s == 0)
    def _():
        pltpu.sync_copy(acc_ul, out_ul.at[c])
        pltpu.sync_copy(acc_uh, out_uh.at[c])


_HALF_U = NU // 2
_HALF_I = NI // 2


@functools.partial(
    pl.kernel,
    out_type=jax.ShapeDtypeStruct((4, NE), _f32),
    mesh=_MESH,
    compiler_params=_SC_PARAMS,
    scratch_types=[
        pltpu.VMEM((_HALF_U,), _i32),
        pltpu.VMEM((_HALF_I,), _i32),
        pltpu.VMEM((CE,), _i32),
        pltpu.VMEM((CE,), _i32),
        pltpu.VMEM((CE,), _f32),
        pltpu.VMEM((CE,), _f32),
        pltpu.VMEM((CE,), _f32),
        pltpu.VMEM((CE,), _f32),
        pltpu.VMEM((CE,), _f32),
    ],
)
def _pass_out(r1, c1, v1, uwq, iwq, out,
              uwq_v, iwq_v, ridx_v, cidx_v, val_v, o1_v, o2_v, o3_v, o4_v):
    base = _wid() * EPT

    pltpu.sync_copy(uwq, uwq_v)
    pltpu.sync_copy(iwq, iwq_v)

    def unpack(table, idx, half):
        lo = idx < half
        word = plsc.load_gather(table, [jnp.where(lo, idx, idx - half)])
        bits = jnp.where(lo, word << 16, word & jnp.int32(-65536))
        return plsc.bitcast(bits, _f32)

    def body(k, carry):
        e0 = base + k * CE
        pltpu.sync_copy(r1.at[pl.ds(e0, CE)], ridx_v)
        pltpu.sync_copy(c1.at[pl.ds(e0, CE)], cidx_v)
        pltpu.sync_copy(v1.at[pl.ds(e0, CE)], val_v)
        for r in range(NVEC):
            sl = pl.ds(r * L, L)
            uw = unpack(uwq_v, ridx_v[sl], _HALF_U)
            iw = unpack(iwq_v, cidx_v[sl], _HALF_I)
            v = val_v[sl]
            av = v * uw
            bv = v - av
            o1 = av * iw
            o4 = bv * iw
            o1_v[sl] = o1
            o2_v[sl] = bv - o4
            o3_v[sl] = av - o1
            o4_v[sl] = o4
        pltpu.sync_copy(o1_v, out.at[0, pl.ds(e0, CE)])
        pltpu.sync_copy(o2_v, out.at[1, pl.ds(e0, CE)])
        pltpu.sync_copy(o3_v, out.at[2, pl.ds(e0, CE)])
        pltpu.sync_copy(o4_v, out.at[3, pl.ds(e0, CE)])
        return carry

    lax.fori_loop(0, NCHUNK, body, 0)


def _rn_bits(x):
    return lax.bitcast_convert_type(x, _i32) + jnp.int32(0x8000)


def _tc_log_counts(counts_p):
    def body(cp, lc):
        lc[...] = jnp.log1p(cp[0:8] + cp[8:16])

    return pl.pallas_call(
        body, out_shape=jax.ShapeDtypeStruct((8, 12500), _f32))(counts_p)


def _tc_pop_pack(pop_p):
    def body(pp, gp):
        pr = pp[0:8] + pp[8:16]
        nrm = jnp.sqrt(jnp.sum(pr * pr))
        pop = pr / (nrm + 1e-8)
        g = 1.0 / jnp.log1p(pop + 1e-8)
        p = 1.0 + pop
        gp[...] = (_rn_bits(p) & jnp.int32(-65536)) | (
            (_rn_bits(g) >> 16) & jnp.int32(0xFFFF))

    return pl.pallas_call(
        body, out_shape=jax.ShapeDtypeStruct((8, 12500), _i32))(pop_p)


def _tc_act(act_p):
    def body(ap, a):
        ar = ap[0:8] + ap[8:16]
        nrm = jnp.sqrt(jnp.sum(ar * ar))
        a[...] = 1.0 + ar / (nrm + 1e-8)

    return pl.pallas_call(
        body, out_shape=jax.ShapeDtypeStruct((8, 12500), _f32))(act_p)


def _tc_weights(ul_p, uh_p, il_p, ih_p):
    def pack_halves(w):
        lo = w[0:4]
        hi = w[4:8]
        return (_rn_bits(hi) & jnp.int32(-65536)) | (
            (_rn_bits(lo) >> 16) & jnp.int32(0xFFFF))

    def weight(low_p, high_p, denom):
        low = (low_p[0:8] + low_p[8:16]) * (1.0 / denom)
        high = jnp.sqrt(high_p[0:8] + high_p[8:16] + 1e-12)
        al = jnp.maximum(low, 1e-6)
        ah = jnp.maximum(high, 1e-6)
        return al / (al + ah)

    def body(ulp, uhp, ilp, ihp, uwq, iwq):
        uwq[...] = pack_halves(weight(ulp, uhp, float(NI)))
        iwq[...] = pack_halves(weight(ilp, ihp, float(NU)))

    return pl.pallas_call(
        body,
        out_shape=[
            jax.ShapeDtypeStruct((4, 12500), _i32),
            jax.ShapeDtypeStruct((4, 12500), _i32),
        ])(ul_p, uh_p, il_p, ih_p)


def kernel(values, row_idx, col_idx):
    f = values.astype(_f32)
    zeros = jnp.zeros((NI,), _f32)
    ones1 = jnp.ones((CE,), _f32)

    counts_p = _pass_counts(col_idx, ones1, zeros)
    lc = _tc_log_counts(counts_p.reshape(16, 12500))

    pop_p = _pass_pop(col_idx, f, lc.reshape(NI), zeros)
    gp = _tc_pop_pack(pop_p.reshape(16, 12500))

    act_p, il_p, ih_p = _pass_act_item(col_idx, row_idx, f, gp.reshape(NI),
                                       zeros)
    a = _tc_act(act_p.reshape(16, 12500))

    ul_p, uh_p = _pass_user(row_idx, f, a.reshape(NU), zeros)
    uwq, iwq = _tc_weights(
        ul_p.reshape(16, 12500), uh_p.reshape(16, 12500),
        il_p.reshape(16, 12500), ih_p.reshape(16, 12500))

    return _pass_out(row_idx, col_idx, f, uwq.reshape(_HALF_U),
                     iwq.reshape(_HALF_I))


# speedup vs baseline: 198.3453x; 1.8288x over previous
import functools

import jax
import jax.numpy as jnp
from jax import lax
from jax.experimental import pallas as pl
from jax.experimental.pallas import tpu as pltpu
from jax.experimental.pallas import tpu_sc as plsc

NU = 100000
NI = 100000
NE = 3200000
NC, NS, L = 2, 16, 16
NW = NC * NS
EPT = NE // NW
CE = 2000
NCHUNK = EPT // CE
NVEC = CE // L
CE_C = 800
NCHUNK_C = EPT // CE_C
NVEC_C = CE_C // L

_MESH = plsc.VectorSubcoreMesh(core_axis_name="c", subcore_axis_name="s")
_SC_PARAMS = pltpu.CompilerParams(needs_layout_passes=False, use_tc_tiling_on_sc=False)

_f32 = jnp.float32
_i32 = jnp.int32


def _wid():
    return lax.axis_index("s") * NC + lax.axis_index("c")


@functools.partial(
    pl.kernel,
    out_type=jax.ShapeDtypeStruct((NC, NI), _f32),
    mesh=_MESH,
    compiler_params=_SC_PARAMS,
    scratch_types=[
        pltpu.VMEM((CE,), _i32),
        pltpu.VMEM((CE,), _f32),
        pltpu.VMEM_SHARED((NI,), _f32),
    ],
)
def _pass_counts(c1, ones1, zeros, out, idx_v, ones_v, acc):
    c = lax.axis_index("c")
    s = lax.axis_index("s")
    base = _wid() * EPT

    @pl.when(s == 0)
    def _():
        pltpu.sync_copy(zeros, acc)

    pltpu.sync_copy(ones1, ones_v)
    plsc.subcore_barrier()

    def body(k, carry):
        e0 = base + k * CE
        pltpu.sync_copy(c1.at[pl.ds(e0, CE)], idx_v)
        pltpu.sync_copy(ones_v, acc.at[idx_v], add=True)
        return carry

    lax.fori_loop(0, NCHUNK, body, 0)
    plsc.subcore_barrier()

    @pl.when(s == 0)
    def _():
        pltpu.sync_copy(acc, out.at[c])


@functools.partial(
    pl.kernel,
    out_type=jax.ShapeDtypeStruct((NC, NI), _f32),
    mesh=_MESH,
    compiler_params=_SC_PARAMS,
    scratch_types=[
        pltpu.VMEM((NI,), _f32),
        pltpu.VMEM((CE,), _i32),
        pltpu.VMEM((CE,), _f32),
        pltpu.VMEM((CE,), _f32),
        pltpu.VMEM_SHARED((NI,), _f32),
    ],
)
def _pass_pop(c1, v1, lc, zeros, out, lc_v, idx_v, val_v, res_v, acc):
    c = lax.axis_index("c")
    s = lax.axis_index("s")
    base = _wid() * EPT

    @pl.when(s == 0)
    def _():
        pltpu.sync_copy(zeros, acc)

    pltpu.sync_copy(lc, lc_v)
    plsc.subcore_barrier()

    def body(k, carry):
        e0 = base + k * CE
        pltpu.sync_copy(c1.at[pl.ds(e0, CE)], idx_v)
        pltpu.sync_copy(v1.at[pl.ds(e0, CE)], val_v)
        for r in range(NVEC):
            sl = pl.ds(r * L, L)
            g = plsc.load_gather(lc_v, [idx_v[sl]])
            res_v[sl] = val_v[sl] * g
        pltpu.sync_copy(res_v, acc.at[idx_v], add=True)
        return carry

    lax.fori_loop(0, NCHUNK, body, 0)
    plsc.subcore_barrier()

    @pl.when(s == 0)
    def _():
        pltpu.sync_copy(acc, out.at[c])


@functools.partial(
    pl.kernel,
    out_type=[
        jax.ShapeDtypeStruct((NC, NU), _f32),
        jax.ShapeDtypeStruct((NC, NI), _f32),
        jax.ShapeDtypeStruct((NC, NI), _f32),
    ],
    mesh=_MESH,
    compiler_params=_SC_PARAMS,
    scratch_types=[
        pltpu.VMEM((NI,), _i32),
        pltpu.VMEM((CE_C,), _i32),
        pltpu.VMEM((CE_C,), _i32),
        pltpu.VMEM((CE_C,), _f32),
        pltpu.VMEM((CE_C,), _f32),
        pltpu.VMEM((CE_C,), _f32),
        pltpu.VMEM((CE_C,), _f32),
        pltpu.VMEM_SHARED((NU,), _f32),
        pltpu.VMEM_SHARED((NI,), _f32),
        pltpu.VMEM_SHARED((NI,), _f32),
    ],
)
def _pass_act_item(c1, r1, v1, gp, zeros, out_act, out_il, out_ih,
                   gp_v, cidx_v, ridx_v, val_v, res_e, res_t, res_t2,
                   acc_act, acc_il, acc_ih):
    c = lax.axis_index("c")
    s = lax.axis_index("s")
    base = _wid() * EPT

    @pl.when(s == 0)
    def _():
        pltpu.sync_copy(zeros, acc_act)
        pltpu.sync_copy(zeros, acc_il)
        pltpu.sync_copy(zeros, acc_ih)

    pltpu.sync_copy(gp, gp_v)
    plsc.subcore_barrier()

    def body(k, carry):
        e0 = base + k * CE_C
        pltpu.sync_copy(c1.at[pl.ds(e0, CE_C)], cidx_v)
        pltpu.sync_copy(r1.at[pl.ds(e0, CE_C)], ridx_v)
        pltpu.sync_copy(v1.at[pl.ds(e0, CE_C)], val_v)
        for r in range(NVEC_C):
            sl = pl.ds(r * L, L)
            q = plsc.load_gather(gp_v, [cidx_v[sl]])
            g = plsc.bitcast(q << 16, _f32)
            p = plsc.bitcast(q & jnp.int32(-65536), _f32)
            v = val_v[sl]
            res_e[sl] = v * g
            t = v * p
            res_t[sl] = t
            res_t2[sl] = t * t
        pltpu.sync_copy(res_e, acc_act.at[ridx_v], add=True)
        pltpu.sync_copy(res_t, acc_il.at[cidx_v], add=True)
        pltpu.sync_copy(res_t2, acc_ih.at[cidx_v], add=True)
        return carry

    lax.fori_loop(0, NCHUNK_C, body, 0)
    plsc.subcore_barrier()

    @pl.when(s == 0)
    def _():
        pltpu.sync_copy(acc_act, out_act.at[c])
        pltpu.sync_copy(acc_il, out_il.at[c])
        pltpu.sync_copy(acc_ih, out_ih.at[c])


@functools.partial(
    pl.kernel,
    out_type=[
        jax.ShapeDtypeStruct((NC, NU), _f32),
        jax.ShapeDtypeStruct((NC, NU), _f32),
    ],
    mesh=_MESH,
    compiler_params=_SC_PARAMS,
    scratch_types=[
        pltpu.VMEM((NU,), _f32),
        pltpu.VMEM((CE,), _i32),
        pltpu.VMEM((CE,), _f32),
        pltpu.VMEM((CE,), _f32),
        pltpu.VMEM((CE,), _f32),
        pltpu.VMEM_SHARED((NU,), _f32),
        pltpu.VMEM_SHARED((NU,), _f32),
    ],
)
def _pass_user(r1, v1, a, zeros, out_ul, out_uh,
               a_v, ridx_v, val_v, res_s, res_s2, acc_ul, acc_uh):
    c = lax.axis_index("c")
    s = lax.axis_index("s")
    base = _wid() * EPT

    @pl.when(s == 0)
    def _():
        pltpu.sync_copy(zeros, acc_ul)
        pltpu.sync_copy(zeros, acc_uh)

    pltpu.sync_copy(a, a_v)
    plsc.subcore_barrier()

    def body(k, carry):
        e0 = base + k * CE
        pltpu.sync_copy(r1.at[pl.ds(e0, CE)], ridx_v)
        pltpu.sync_copy(v1.at[pl.ds(e0, CE)], val_v)
        for r in range(NVEC):
            sl = pl.ds(r * L, L)
            av = plsc.load_gather(a_v, [ridx_v[sl]])
            sg = val_v[sl] * av
            res_s[sl] = sg
            res_s2[sl] = sg * sg
        pltpu.sync_copy(res_s, acc_ul.at[ridx_v], add=True)
        pltpu.sync_copy(res_s2, acc_uh.at[ridx_v], add=True)
        return carry

    lax.fori_loop(0, NCHUNK, body, 0)
    plsc.subcore_barrier()

    @pl.when(s == 0)
    def _():
        pltpu.sync_copy(acc_ul, out_ul.at[c])
        pltpu.sync_copy(acc_uh, out_uh.at[c])


_HALF_U = NU // 2
_HALF_I = NI // 2


@functools.partial(
    pl.kernel,
    out_type=[
        jax.ShapeDtypeStruct((NE,), _f32),
        jax.ShapeDtypeStruct((NE,), _f32),
        jax.ShapeDtypeStruct((NE,), _f32),
        jax.ShapeDtypeStruct((NE,), _f32),
    ],
    mesh=_MESH,
    compiler_params=_SC_PARAMS,
    scratch_types=[
        pltpu.VMEM((_HALF_U,), _i32),
        pltpu.VMEM((_HALF_I,), _i32),
        pltpu.VMEM((CE,), _i32),
        pltpu.VMEM((CE,), _i32),
        pltpu.VMEM((CE,), _f32),
        pltpu.VMEM((CE,), _f32),
        pltpu.VMEM((CE,), _f32),
        pltpu.VMEM((CE,), _f32),
        pltpu.VMEM((CE,), _f32),
    ],
)
def _pass_out(r1, c1, v1, uwq, iwq, out1, out2, out3, out4,
              uwq_v, iwq_v, ridx_v, cidx_v, val_v, o1_v, o2_v, o3_v, o4_v):
    base = _wid() * EPT

    pltpu.sync_copy(uwq, uwq_v)
    pltpu.sync_copy(iwq, iwq_v)

    def unpack(table, idx, half):
        lo = idx < half
        word = plsc.load_gather(table, [jnp.where(lo, idx, idx - half)])
        bits = jnp.where(lo, word << 16, word & jnp.int32(-65536))
        return plsc.bitcast(bits, _f32)

    def body(k, carry):
        e0 = base + k * CE
        pltpu.sync_copy(r1.at[pl.ds(e0, CE)], ridx_v)
        pltpu.sync_copy(c1.at[pl.ds(e0, CE)], cidx_v)
        pltpu.sync_copy(v1.at[pl.ds(e0, CE)], val_v)
        for r in range(NVEC):
            sl = pl.ds(r * L, L)
            uw = unpack(uwq_v, ridx_v[sl], _HALF_U)
            iw = unpack(iwq_v, cidx_v[sl], _HALF_I)
            v = val_v[sl]
            av = v * uw
            bv = v - av
            o1 = av * iw
            o4 = bv * iw
            o1_v[sl] = o1
            o2_v[sl] = bv - o4
            o3_v[sl] = av - o1
            o4_v[sl] = o4
        pltpu.sync_copy(o1_v, out1.at[pl.ds(e0, CE)])
        pltpu.sync_copy(o2_v, out2.at[pl.ds(e0, CE)])
        pltpu.sync_copy(o3_v, out3.at[pl.ds(e0, CE)])
        pltpu.sync_copy(o4_v, out4.at[pl.ds(e0, CE)])
        return carry

    lax.fori_loop(0, NCHUNK, body, 0)


def _rn_bits(x):
    return lax.bitcast_convert_type(x, _i32) + jnp.int32(0x8000)


def _tc_log_counts(counts_p):
    def body(cp, lc):
        lc[...] = jnp.log1p(cp[0:8] + cp[8:16])

    return pl.pallas_call(
        body, out_shape=jax.ShapeDtypeStruct((8, 12500), _f32))(counts_p)


def _tc_pop_pack(pop_p):
    def body(pp, gp):
        pr = pp[0:8] + pp[8:16]
        nrm = jnp.sqrt(jnp.sum(pr * pr))
        pop = pr / (nrm + 1e-8)
        g = 1.0 / jnp.log1p(pop + 1e-8)
        p = 1.0 + pop
        gp[...] = (_rn_bits(p) & jnp.int32(-65536)) | (
            (_rn_bits(g) >> 16) & jnp.int32(0xFFFF))

    return pl.pallas_call(
        body, out_shape=jax.ShapeDtypeStruct((8, 12500), _i32))(pop_p)


def _tc_act(act_p):
    def body(ap, a):
        ar = ap[0:8] + ap[8:16]
        nrm = jnp.sqrt(jnp.sum(ar * ar))
        a[...] = 1.0 + ar / (nrm + 1e-8)

    return pl.pallas_call(
        body, out_shape=jax.ShapeDtypeStruct((8, 12500), _f32))(act_p)


def _tc_weights(ul_p, uh_p, il_p, ih_p):
    def pack_halves(w):
        lo = w[0:4]
        hi = w[4:8]
        return (_rn_bits(hi) & jnp.int32(-65536)) | (
            (_rn_bits(lo) >> 16) & jnp.int32(0xFFFF))

    def weight(low_p, high_p, denom):
        low = (low_p[0:8] + low_p[8:16]) * (1.0 / denom)
        high = jnp.sqrt(high_p[0:8] + high_p[8:16] + 1e-12)
        al = jnp.maximum(low, 1e-6)
        ah = jnp.maximum(high, 1e-6)
        return al / (al + ah)

    def body(ulp, uhp, ilp, ihp, uwq, iwq):
        uwq[...] = pack_halves(weight(ulp, uhp, float(NI)))
        iwq[...] = pack_halves(weight(ilp, ihp, float(NU)))

    return pl.pallas_call(
        body,
        out_shape=[
            jax.ShapeDtypeStruct((4, 12500), _i32),
            jax.ShapeDtypeStruct((4, 12500), _i32),
        ])(ul_p, uh_p, il_p, ih_p)


def kernel(values, row_idx, col_idx):
    f = values.astype(_f32)
    zeros = jnp.zeros((NI,), _f32)
    ones1 = jnp.ones((CE,), _f32)

    counts_p = _pass_counts(col_idx, ones1, zeros)
    lc = _tc_log_counts(counts_p.reshape(16, 12500))

    pop_p = _pass_pop(col_idx, f, lc.reshape(NI), zeros)
    gp = _tc_pop_pack(pop_p.reshape(16, 12500))

    act_p, il_p, ih_p = _pass_act_item(col_idx, row_idx, f, gp.reshape(NI),
                                       zeros)
    a = _tc_act(act_p.reshape(16, 12500))

    ul_p, uh_p = _pass_user(row_idx, f, a.reshape(NU), zeros)
    uwq, iwq = _tc_weights(
        ul_p.reshape(16, 12500), uh_p.reshape(16, 12500),
        il_p.reshape(16, 12500), ih_p.reshape(16, 12500))

    o1, o2, o3, o4 = _pass_out(row_idx, col_idx, f, uwq.reshape(_HALF_U),
                               iwq.reshape(_HALF_I))
    return jnp.stack([o1, o2, o3, o4], axis=0)


# speedup vs baseline: 368.6322x; 1.8585x over previous
import functools

import jax
import jax.numpy as jnp
from jax import lax
from jax.experimental import pallas as pl
from jax.experimental.pallas import tpu as pltpu
from jax.experimental.pallas import tpu_sc as plsc

NU = 100000
NI = 100000
NE = 3200000
NC, NS, L = 2, 16, 16
NW = NC * NS
EPT = NE // NW

CE_A = 10000
NCH_A = EPT // CE_A
CE_B = 2000
NCH_B = EPT // CE_B
NV_B = CE_B // L
CE_C = 800
NCH_C = EPT // CE_C
NV_C = CE_C // L
CE_D = 2000
NCH_D = EPT // CE_D
NV_D = CE_D // L
CE_E = 2000
NCH_E = EPT // CE_E
NV_E = CE_E // L

_MESH = plsc.VectorSubcoreMesh(core_axis_name="c", subcore_axis_name="s")
_SC_PARAMS = pltpu.CompilerParams(needs_layout_passes=False,
                                  use_tc_tiling_on_sc=False)

_f32 = jnp.float32
_i32 = jnp.int32


def _wid():
    return lax.axis_index("s") * NC + lax.axis_index("c")


def _db3_loop(nchunk, issue_load, wait_load, process, wait_scatter):
    issue_load(0, 0)

    def traced_step(k, b3, b2):
        @pl.when(k + 1 < nchunk)
        def _():
            @pl.when(k >= 2)
            def _():
                wait_scatter((b3 + 1) % 3, b2)
            issue_load(k + 1, (b3 + 1) % 3)
        wait_load(k, b3)
        process(k, b3, b2)

    nfull = nchunk // 6

    def outer(ko, carry):
        k0 = ko * 6
        for b in range(6):
            traced_step(k0 + b, b % 3, b % 2)
        return carry

    lax.fori_loop(0, nfull, outer, 0)
    for k in range(nfull * 6, nchunk):
        if k + 1 < nchunk:
            if k >= 2:
                wait_scatter((k + 1) % 3, k % 2)
            issue_load(k + 1, (k + 1) % 3)
        wait_load(k, k % 3)
        process(k, k % 3, k % 2)
    for j in range(max(0, nchunk - 3), nchunk):
        wait_scatter(j % 3, j % 2)


@functools.partial(
    pl.kernel,
    out_type=jax.ShapeDtypeStruct((NC, NI), _f32),
    mesh=_MESH,
    compiler_params=_SC_PARAMS,
    scratch_types=[
        pltpu.VMEM((CE_A,), _i32),
        pltpu.VMEM((CE_A,), _i32),
        pltpu.VMEM((CE_A,), _i32),
        pltpu.VMEM((CE_A,), _f32),
        pltpu.VMEM_SHARED((NI,), _f32),
        pltpu.SemaphoreType.DMA,
        pltpu.SemaphoreType.DMA,
        pltpu.SemaphoreType.DMA,
        pltpu.SemaphoreType.DMA,
        pltpu.SemaphoreType.DMA,
    ],
)
def _pass_counts(c1, ones1, zeros, out, idx0, idx1, idx2, ones_v, acc,
                 seml0, seml1, seml2, sems0, sems1):
    c = lax.axis_index("c")
    s = lax.axis_index("s")
    base = _wid() * EPT
    idx = (idx0, idx1, idx2)
    seml = (seml0, seml1, seml2)
    sems = (sems0, sems1)

    @pl.when(s == 0)
    def _():
        pltpu.sync_copy(zeros, acc)

    pltpu.sync_copy(ones1, ones_v)
    plsc.subcore_barrier()

    def issue(k, b3):
        e0 = base + k * CE_A
        pltpu.async_copy(c1.at[pl.ds(e0, CE_A)], idx[b3], seml[b3])

    def wait(k, b3):
        e0 = base + k * CE_A
        pltpu.make_async_copy(c1.at[pl.ds(e0, CE_A)], idx[b3],
                              seml[b3]).wait()

    def process(k, b3, b2):
        pltpu.async_copy(ones_v, acc.at[idx[b3]], sems[b2], add=True)

    def wait_scatter(b3, b2):
        pltpu.make_async_copy(ones_v, acc.at[idx[b3]], sems[b2]).wait()

    _db3_loop(NCH_A, issue, wait, process, wait_scatter)
    plsc.subcore_barrier()

    @pl.when(s == 0)
    def _():
        pltpu.sync_copy(acc, out.at[c])


@functools.partial(
    pl.kernel,
    out_type=jax.ShapeDtypeStruct((NC, NI), _f32),
    mesh=_MESH,
    compiler_params=_SC_PARAMS,
    scratch_types=[
        pltpu.VMEM((NI,), _f32),
        pltpu.VMEM((CE_B,), _i32),
        pltpu.VMEM((CE_B,), _i32),
        pltpu.VMEM((CE_B,), _i32),
        pltpu.VMEM((CE_B,), _f32),
        pltpu.VMEM((CE_B,), _f32),
        pltpu.VMEM((CE_B,), _f32),
        pltpu.VMEM_SHARED((NI,), _f32),
        pltpu.SemaphoreType.DMA,
        pltpu.SemaphoreType.DMA,
        pltpu.SemaphoreType.DMA,
        pltpu.SemaphoreType.DMA,
        pltpu.SemaphoreType.DMA,
    ],
)
def _pass_pop(c1, v1, lc, zeros, out, lc_v, idx0, idx1, idx2,
              val0, val1, val2, acc, seml0, seml1, seml2, sems0, sems1):
    c = lax.axis_index("c")
    s = lax.axis_index("s")
    base = _wid() * EPT
    idx = (idx0, idx1, idx2)
    val = (val0, val1, val2)
    seml = (seml0, seml1, seml2)
    sems = (sems0, sems1)

    @pl.when(s == 0)
    def _():
        pltpu.sync_copy(zeros, acc)

    pltpu.sync_copy(lc, lc_v)
    plsc.subcore_barrier()

    def issue(k, b3):
        e0 = base + k * CE_B
        pltpu.async_copy(c1.at[pl.ds(e0, CE_B)], idx[b3], seml[b3])
        pltpu.async_copy(v1.at[pl.ds(e0, CE_B)], val[b3], seml[b3])

    def wait(k, b3):
        e0 = base + k * CE_B
        pltpu.make_async_copy(c1.at[pl.ds(e0, CE_B)], idx[b3],
                              seml[b3]).wait()
        pltpu.make_async_copy(v1.at[pl.ds(e0, CE_B)], val[b3],
                              seml[b3]).wait()

    def process(k, b3, b2):
        for r in range(NV_B):
            sl = pl.ds(r * L, L)
            g = plsc.load_gather(lc_v, [idx[b3][sl]])
            val[b3][sl] = val[b3][sl] * g
        pltpu.async_copy(val[b3], acc.at[idx[b3]], sems[b2], add=True)

    def wait_scatter(b3, b2):
        pltpu.make_async_copy(val[b3], acc.at[idx[b3]], sems[b2]).wait()

    _db3_loop(NCH_B, issue, wait, process, wait_scatter)
    plsc.subcore_barrier()

    @pl.when(s == 0)
    def _():
        pltpu.sync_copy(acc, out.at[c])


@functools.partial(
    pl.kernel,
    out_type=[
        jax.ShapeDtypeStruct((NC, NU), _f32),
        jax.ShapeDtypeStruct((NC, NI), _f32),
        jax.ShapeDtypeStruct((NC, NI), _f32),
    ],
    mesh=_MESH,
    compiler_params=_SC_PARAMS,
    scratch_types=[
        pltpu.VMEM((NI,), _i32),
        pltpu.VMEM((CE_C,), _i32),
        pltpu.VMEM((CE_C,), _i32),
        pltpu.VMEM((CE_C,), _i32),
        pltpu.VMEM((CE_C,), _i32),
        pltpu.VMEM((CE_C,), _i32),
        pltpu.VMEM((CE_C,), _i32),
        pltpu.VMEM((CE_C,), _f32),
        pltpu.VMEM((CE_C,), _f32),
        pltpu.VMEM((CE_C,), _f32),
        pltpu.VMEM((CE_C,), _f32),
        pltpu.VMEM((CE_C,), _f32),
        pltpu.VMEM((CE_C,), _f32),
        pltpu.VMEM((CE_C,), _f32),
        pltpu.VMEM_SHARED((NU,), _f32),
        pltpu.VMEM_SHARED((NI,), _f32),
        pltpu.VMEM_SHARED((NI,), _f32),
        pltpu.SemaphoreType.DMA,
        pltpu.SemaphoreType.DMA,
        pltpu.SemaphoreType.DMA,
        pltpu.SemaphoreType.DMA,
        pltpu.SemaphoreType.DMA,
    ],
)
def _pass_act_item(c1, r1, v1, gp, zeros, out_act, out_il, out_ih,
                   gp_v, cidx0, cidx1, cidx2, ridx0, ridx1, ridx2,
                   val0, val1, val2, rt0, rt1, rq0, rq1,
                   acc_act, acc_il, acc_ih,
                   seml0, seml1, seml2, sems0, sems1):
    c = lax.axis_index("c")
    s = lax.axis_index("s")
    base = _wid() * EPT
    cidx = (cidx0, cidx1, cidx2)
    ridx = (ridx0, ridx1, ridx2)
    val = (val0, val1, val2)
    rt = (rt0, rt1)
    rq = (rq0, rq1)
    seml = (seml0, seml1, seml2)
    sems = (sems0, sems1)

    @pl.when(s == 0)
    def _():
        pltpu.sync_copy(zeros, acc_act)
        pltpu.sync_copy(zeros, acc_il)
        pltpu.sync_copy(zeros, acc_ih)

    pltpu.sync_copy(gp, gp_v)
    plsc.subcore_barrier()

    def issue(k, b3):
        e0 = base + k * CE_C
        pltpu.async_copy(c1.at[pl.ds(e0, CE_C)], cidx[b3], seml[b3])
        pltpu.async_copy(r1.at[pl.ds(e0, CE_C)], ridx[b3], seml[b3])
        pltpu.async_copy(v1.at[pl.ds(e0, CE_C)], val[b3], seml[b3])

    def wait(k, b3):
        e0 = base + k * CE_C
        pltpu.make_async_copy(c1.at[pl.ds(e0, CE_C)], cidx[b3],
                              seml[b3]).wait()
        pltpu.make_async_copy(r1.at[pl.ds(e0, CE_C)], ridx[b3],
                              seml[b3]).wait()
        pltpu.make_async_copy(v1.at[pl.ds(e0, CE_C)], val[b3],
                              seml[b3]).wait()

    def process(k, b3, b2):
        for r in range(NV_C):
            sl = pl.ds(r * L, L)
            q = plsc.load_gather(gp_v, [cidx[b3][sl]])
            g = plsc.bitcast(q << 16, _f32)
            p = plsc.bitcast(q & jnp.int32(-65536), _f32)
            v = val[b3][sl]
            t = v * p
            rt[b2][sl] = t
            rq[b2][sl] = t * t
            val[b3][sl] = v * g
        pltpu.async_copy(rt[b2], acc_il.at[cidx[b3]], sems[b2], add=True)
        pltpu.async_copy(rq[b2], acc_ih.at[cidx[b3]], sems[b2], add=True)
        pltpu.async_copy(val[b3], acc_act.at[ridx[b3]], sems[b2], add=True)

    def wait_scatter(b3, b2):
        pltpu.make_async_copy(rt[b2], acc_il.at[cidx[b3]], sems[b2]).wait()
        pltpu.make_async_copy(rq[b2], acc_ih.at[cidx[b3]], sems[b2]).wait()
        pltpu.make_async_copy(val[b3], acc_act.at[ridx[b3]], sems[b2]).wait()

    _db3_loop(NCH_C, issue, wait, process, wait_scatter)
    plsc.subcore_barrier()

    @pl.when(s == 0)
    def _():
        pltpu.sync_copy(acc_act, out_act.at[c])
        pltpu.sync_copy(acc_il, out_il.at[c])
        pltpu.sync_copy(acc_ih, out_ih.at[c])


@functools.partial(
    pl.kernel,
    out_type=[
        jax.ShapeDtypeStruct((NC, NU), _f32),
        jax.ShapeDtypeStruct((NC, NU), _f32),
    ],
    mesh=_MESH,
    compiler_params=_SC_PARAMS,
    scratch_types=[
        pltpu.VMEM((NU,), _f32),
        pltpu.VMEM((CE_D,), _i32),
        pltpu.VMEM((CE_D,), _i32),
        pltpu.VMEM((CE_D,), _i32),
        pltpu.VMEM((CE_D,), _f32),
        pltpu.VMEM((CE_D,), _f32),
        pltpu.VMEM((CE_D,), _f32),
        pltpu.VMEM((CE_D,), _f32),
        pltpu.VMEM((CE_D,), _f32),
        pltpu.VMEM_SHARED((NU,), _f32),
        pltpu.VMEM_SHARED((NU,), _f32),
        pltpu.SemaphoreType.DMA,
        pltpu.SemaphoreType.DMA,
        pltpu.SemaphoreType.DMA,
        pltpu.SemaphoreType.DMA,
        pltpu.SemaphoreType.DMA,
    ],
)
def _pass_user(r1, v1, a, zeros, out_ul, out_uh,
               a_v, ridx0, ridx1, ridx2, val0, val1, val2, rq0, rq1,
               acc_ul, acc_uh, seml0, seml1, seml2, sems0, sems1):
    c = lax.axis_index("c")
    s = lax.axis_index("s")
    base = _wid() * EPT
    ridx = (ridx0, ridx1, ridx2)
    val = (val0, val1, val2)
    rq = (rq0, rq1)
    seml = (seml0, seml1, seml2)
    sems = (sems0, sems1)

    @pl.when(s == 0)
    def _():
        pltpu.sync_copy(zeros, acc_ul)
        pltpu.sync_copy(zeros, acc_uh)

    pltpu.sync_copy(a, a_v)
    plsc.subcore_barrier()

    def issue(k, b3):
        e0 = base + k * CE_D
        pltpu.async_copy(r1.at[pl.ds(e0, CE_D)], ridx[b3], seml[b3])
        pltpu.async_copy(v1.at[pl.ds(e0, CE_D)], val[b3], seml[b3])

    def wait(k, b3):
        e0 = base + k * CE_D
        pltpu.make_async_copy(r1.at[pl.ds(e0, CE_D)], ridx[b3],
                              seml[b3]).wait()
        pltpu.make_async_copy(v1.at[pl.ds(e0, CE_D)], val[b3],
                              seml[b3]).wait()

    def process(k, b3, b2):
        for r in range(NV_D):
            sl = pl.ds(r * L, L)
            av = plsc.load_gather(a_v, [ridx[b3][sl]])
            sg = val[b3][sl] * av
            val[b3][sl] = sg
            rq[b2][sl] = sg * sg
        pltpu.async_copy(val[b3], acc_ul.at[ridx[b3]], sems[b2], add=True)
        pltpu.async_copy(rq[b2], acc_uh.at[ridx[b3]], sems[b2], add=True)

    def wait_scatter(b3, b2):
        pltpu.make_async_copy(val[b3], acc_ul.at[ridx[b3]], sems[b2]).wait()
        pltpu.make_async_copy(rq[b2], acc_uh.at[ridx[b3]], sems[b2]).wait()

    _db3_loop(NCH_D, issue, wait, process, wait_scatter)
    plsc.subcore_barrier()

    @pl.when(s == 0)
    def _():
        pltpu.sync_copy(acc_ul, out_ul.at[c])
        pltpu.sync_copy(acc_uh, out_uh.at[c])


_HALF_U = NU // 2
_HALF_I = NI // 2


@functools.partial(
    pl.kernel,
    out_type=[
        jax.ShapeDtypeStruct((NE,), _f32),
        jax.ShapeDtypeStruct((NE,), _f32),
        jax.ShapeDtypeStruct((NE,), _f32),
        jax.ShapeDtypeStruct((NE,), _f32),
    ],
    mesh=_MESH,
    compiler_params=_SC_PARAMS,
    scratch_types=[
        pltpu.VMEM((_HALF_U,), _i32),
        pltpu.VMEM((_HALF_I,), _i32),
        pltpu.VMEM((CE_E,), _i32),
        pltpu.VMEM((CE_E,), _i32),
        pltpu.VMEM((CE_E,), _i32),
        pltpu.VMEM((CE_E,), _i32),
        pltpu.VMEM((CE_E,), _f32),
        pltpu.VMEM((CE_E,), _f32),
        pltpu.VMEM((CE_E,), _f32),
        pltpu.VMEM((CE_E,), _f32),
        pltpu.VMEM((CE_E,), _f32),
        pltpu.VMEM((CE_E,), _f32),
        pltpu.VMEM((CE_E,), _f32),
        pltpu.VMEM((CE_E,), _f32),
        pltpu.VMEM((CE_E,), _f32),
        pltpu.VMEM((CE_E,), _f32),
        pltpu.SemaphoreType.DMA,
        pltpu.SemaphoreType.DMA,
        pltpu.SemaphoreType.DMA,
        pltpu.SemaphoreType.DMA,
    ],
)
def _pass_out(r1, c1, v1, uwq, iwq, out1, out2, out3, out4,
              uwq_v, iwq_v, ridx0, ridx1, cidx0, cidx1, val0, val1,
              o1a, o1b, o2a, o2b, o3a, o3b, o4a, o4b,
              sem0, sem1, semw0, semw1):
    base = _wid() * EPT
    ridx = (ridx0, ridx1)
    cidx = (cidx0, cidx1)
    val = (val0, val1)
    ov = ((o1a, o2a, o3a, o4a), (o1b, o2b, o3b, o4b))
    outs = (out1, out2, out3, out4)
    sem = (sem0, sem1)
    semw = (semw0, semw1)

    pltpu.sync_copy(uwq, uwq_v)
    pltpu.sync_copy(iwq, iwq_v)

    def unpack(table, idx, half):
        lo = idx < half
        word = plsc.load_gather(table, [jnp.where(lo, idx, idx - half)])
        bits = jnp.where(lo, word << 16, word & jnp.int32(-65536))
        return plsc.bitcast(bits, _f32)

    def issue(k, b):
        e0 = base + k * CE_E
        pltpu.async_copy(r1.at[pl.ds(e0, CE_E)], ridx[b], sem[b])
        pltpu.async_copy(c1.at[pl.ds(e0, CE_E)], cidx[b], sem[b])
        pltpu.async_copy(v1.at[pl.ds(e0, CE_E)], val[b], sem[b])

    def wait(k, b):
        e0 = base + k * CE_E
        pltpu.make_async_copy(r1.at[pl.ds(e0, CE_E)], ridx[b], sem[b]).wait()
        pltpu.make_async_copy(c1.at[pl.ds(e0, CE_E)], cidx[b], sem[b]).wait()
        pltpu.make_async_copy(v1.at[pl.ds(e0, CE_E)], val[b], sem[b]).wait()

    def wait_writes(k, b):
        e0 = base + k * CE_E
        for q in range(4):
            pltpu.make_async_copy(
                ov[b][q], outs[q].at[pl.ds(e0, CE_E)], semw[b]).wait()

    def process(k, b):
        @pl.when(k >= 2)
        def _():
            wait_writes(k - 2, b)
        for r in range(NV_E):
            sl = pl.ds(r * L, L)
            uw = unpack(uwq_v, ridx[b][sl], _HALF_U)
            iw = unpack(iwq_v, cidx[b][sl], _HALF_I)
            v = val[b][sl]
            av = v * uw
            bv = v - av
            o1 = av * iw
            o4 = bv * iw
            ov[b][0][sl] = o1
            ov[b][1][sl] = bv - o4
            ov[b][2][sl] = av - o1
            ov[b][3][sl] = o4
        e0 = base + k * CE_E
        for q in range(4):
            pltpu.async_copy(ov[b][q], outs[q].at[pl.ds(e0, CE_E)], semw[b])

    issue(0, 0)

    def outer(ko, carry):
        for b in range(2):
            k = ko * 2 + b

            @pl.when(k + 1 < NCH_E)
            def _():
                issue(k + 1, 1 - b)

            wait(k, b)
            process(k, b)
        return carry

    lax.fori_loop(0, NCH_E // 2, outer, 0)
    wait_writes(NCH_E - 2, 0)
    wait_writes(NCH_E - 1, 1)


def _rn_bits(x):
    return lax.bitcast_convert_type(x, _i32) + jnp.int32(0x8000)


def _tc_log_counts(counts_p):
    def body(cp, lc):
        lc[...] = jnp.log1p(cp[0:8] + cp[8:16])

    return pl.pallas_call(
        body, out_shape=jax.ShapeDtypeStruct((8, 12500), _f32))(counts_p)


def _tc_pop_pack(pop_p):
    def body(pp, gp):
        pr = pp[0:8] + pp[8:16]
        nrm = jnp.sqrt(jnp.sum(pr * pr))
        pop = pr / (nrm + 1e-8)
        g = 1.0 / jnp.log1p(pop + 1e-8)
        p = 1.0 + pop
        gp[...] = (_rn_bits(p) & jnp.int32(-65536)) | (
            (_rn_bits(g) >> 16) & jnp.int32(0xFFFF))

    return pl.pallas_call(
        body, out_shape=jax.ShapeDtypeStruct((8, 12500), _i32))(pop_p)


def _tc_act(act_p):
    def body(ap, a):
        ar = ap[0:8] + ap[8:16]
        nrm = jnp.sqrt(jnp.sum(ar * ar))
        a[...] = 1.0 + ar / (nrm + 1e-8)

    return pl.pallas_call(
        body, out_shape=jax.ShapeDtypeStruct((8, 12500), _f32))(act_p)


def _tc_weights(ul_p, uh_p, il_p, ih_p):
    def pack_halves(w):
        lo = w[0:4]
        hi = w[4:8]
        return (_rn_bits(hi) & jnp.int32(-65536)) | (
            (_rn_bits(lo) >> 16) & jnp.int32(0xFFFF))

    def weight(low_p, high_p, denom):
        low = (low_p[0:8] + low_p[8:16]) * (1.0 / denom)
        high = jnp.sqrt(high_p[0:8] + high_p[8:16] + 1e-12)
        al = jnp.maximum(low, 1e-6)
        ah = jnp.maximum(high, 1e-6)
        return al / (al + ah)

    def body(ulp, uhp, ilp, ihp, uwq, iwq):
        uwq[...] = pack_halves(weight(ulp, uhp, float(NI)))
        iwq[...] = pack_halves(weight(ilp, ihp, float(NU)))

    return pl.pallas_call(
        body,
        out_shape=[
            jax.ShapeDtypeStruct((4, 12500), _i32),
            jax.ShapeDtypeStruct((4, 12500), _i32),
        ])(ul_p, uh_p, il_p, ih_p)


def kernel(values, row_idx, col_idx):
    f = values.astype(_f32)
    zeros = jnp.zeros((NI,), _f32)
    ones1 = jnp.ones((CE_A,), _f32)

    counts_p = _pass_counts(col_idx, ones1, zeros)
    lc = _tc_log_counts(counts_p.reshape(16, 12500))

    pop_p = _pass_pop(col_idx, f, lc.reshape(NI), zeros)
    gp = _tc_pop_pack(pop_p.reshape(16, 12500))

    act_p, il_p, ih_p = _pass_act_item(col_idx, row_idx, f, gp.reshape(NI),
                                       zeros)
    a = _tc_act(act_p.reshape(16, 12500))

    ul_p, uh_p = _pass_user(row_idx, f, a.reshape(NU), zeros)
    uwq, iwq = _tc_weights(
        ul_p.reshape(16, 12500), uh_p.reshape(16, 12500),
        il_p.reshape(16, 12500), ih_p.reshape(16, 12500))

    o1, o2, o3, o4 = _pass_out(row_idx, col_idx, f, uwq.reshape(_HALF_U),
                               iwq.reshape(_HALF_I))
    return jnp.stack([o1, o2, o3, o4], axis=0)


# speedup vs baseline: 459.9712x; 1.2478x over previous
import functools

import jax
import jax.numpy as jnp
from jax import lax
from jax.experimental import pallas as pl
from jax.experimental.pallas import tpu as pltpu
from jax.experimental.pallas import tpu_sc as plsc

NU = 100000
NI = 100000
NE = 3200000
NC, NS, L = 2, 16, 16
NW = NC * NS
EPT = NE // NW

CE_A = 10000
NCH_A = EPT // CE_A
CE_B = 2000
NCH_B = EPT // CE_B
NV_B = CE_B // L
CE_C = 800
NCH_C = EPT // CE_C
NV_C = CE_C // L
CE_D = 2000
NCH_D = EPT // CE_D
NV_D = CE_D // L
CE_E = 2000
NCH_E = EPT // CE_E
NV_E = CE_E // L

_MESH = plsc.VectorSubcoreMesh(core_axis_name="c", subcore_axis_name="s")
_SC_PARAMS = pltpu.CompilerParams(needs_layout_passes=False,
                                  use_tc_tiling_on_sc=False)

_f32 = jnp.float32
_i32 = jnp.int32


def _wid():
    return lax.axis_index("s") * NC + lax.axis_index("c")


def _db3_loop(nchunk, issue_load, wait_load, process, wait_scatter):
    issue_load(0, 0)

    def traced_step(k, b3, b2):
        @pl.when(k + 1 < nchunk)
        def _():
            @pl.when(k >= 2)
            def _():
                wait_scatter((b3 + 1) % 3, b2)
            issue_load(k + 1, (b3 + 1) % 3)
        wait_load(k, b3)
        process(k, b3, b2)

    nfull = nchunk // 6

    def outer(ko, carry):
        k0 = ko * 6
        for b in range(6):
            traced_step(k0 + b, b % 3, b % 2)
        return carry

    lax.fori_loop(0, nfull, outer, 0)
    for k in range(nfull * 6, nchunk):
        if k + 1 < nchunk:
            if k >= 2:
                wait_scatter((k + 1) % 3, k % 2)
            issue_load(k + 1, (k + 1) % 3)
        wait_load(k, k % 3)
        process(k, k % 3, k % 2)
    for j in range(max(0, nchunk - 3), nchunk):
        wait_scatter(j % 3, j % 2)


CE_S = 4000
NCH_S = EPT // CE_S
NV_S = CE_S // L


@functools.partial(
    pl.kernel,
    out_type=[
        jax.ShapeDtypeStruct((NC, NI), _f32),
        jax.ShapeDtypeStruct((NC, NI), _f32),
        jax.ShapeDtypeStruct((NC, NI), _f32),
        jax.ShapeDtypeStruct((NC, NU), _f32),
        jax.ShapeDtypeStruct((NC, NU), _f32),
    ],
    mesh=_MESH,
    compiler_params=_SC_PARAMS,
    scratch_types=[
        pltpu.VMEM((CE_S,), _i32),
        pltpu.VMEM((CE_S,), _i32),
        pltpu.VMEM((CE_S,), _i32),
        pltpu.VMEM((CE_S,), _i32),
        pltpu.VMEM((CE_S,), _i32),
        pltpu.VMEM((CE_S,), _i32),
        pltpu.VMEM((CE_S,), _f32),
        pltpu.VMEM((CE_S,), _f32),
        pltpu.VMEM((CE_S,), _f32),
        pltpu.VMEM((CE_S,), _f32),
        pltpu.VMEM((CE_S,), _f32),
        pltpu.VMEM((CE_S,), _f32),
        pltpu.VMEM_SHARED((NI,), _f32),
        pltpu.VMEM_SHARED((NI,), _f32),
        pltpu.VMEM_SHARED((NI,), _f32),
        pltpu.VMEM_SHARED((NU,), _f32),
        pltpu.VMEM_SHARED((NU,), _f32),
        pltpu.SemaphoreType.DMA,
        pltpu.SemaphoreType.DMA,
        pltpu.SemaphoreType.DMA,
        pltpu.SemaphoreType.DMA,
        pltpu.SemaphoreType.DMA,
    ],
)
def _pass_stats(c1, r1, v1, ones1, zeros,
                out_cnt, out_sfc, out_sf2c, out_sfr, out_sf2r,
                cidx0, cidx1, cidx2, ridx0, ridx1, ridx2,
                val0, val1, val2, ones_v, rq0, rq1,
                acc_cnt, acc_sfc, acc_sf2c, acc_sfr, acc_sf2r,
                seml0, seml1, seml2, sems0, sems1):
    c = lax.axis_index("c")
    s = lax.axis_index("s")
    base = _wid() * EPT
    cidx = (cidx0, cidx1, cidx2)
    ridx = (ridx0, ridx1, ridx2)
    val = (val0, val1, val2)
    rq = (rq0, rq1)
    seml = (seml0, seml1, seml2)
    sems = (sems0, sems1)

    @pl.when(s == 0)
    def _():
        pltpu.sync_copy(zeros, acc_cnt)
        pltpu.sync_copy(zeros, acc_sfc)
        pltpu.sync_copy(zeros, acc_sf2c)
        pltpu.sync_copy(zeros.at[pl.ds(0, NU)], acc_sfr)
        pltpu.sync_copy(zeros.at[pl.ds(0, NU)], acc_sf2r)

    pltpu.sync_copy(ones1, ones_v)
    plsc.subcore_barrier()

    def issue(k, b3):
        e0 = base + k * CE_S
        pltpu.async_copy(c1.at[pl.ds(e0, CE_S)], cidx[b3], seml[b3])
        pltpu.async_copy(r1.at[pl.ds(e0, CE_S)], ridx[b3], seml[b3])
        pltpu.async_copy(v1.at[pl.ds(e0, CE_S)], val[b3], seml[b3])

    def wait(k, b3):
        e0 = base + k * CE_S
        pltpu.make_async_copy(c1.at[pl.ds(e0, CE_S)], cidx[b3],
                              seml[b3]).wait()
        pltpu.make_async_copy(r1.at[pl.ds(e0, CE_S)], ridx[b3],
                              seml[b3]).wait()
        pltpu.make_async_copy(v1.at[pl.ds(e0, CE_S)], val[b3],
                              seml[b3]).wait()

    def process(k, b3, b2):
        for r in range(NV_S):
            sl = pl.ds(r * L, L)
            v = val[b3][sl]
            rq[b2][sl] = v * v
        pltpu.async_copy(ones_v, acc_cnt.at[cidx[b3]], sems[b2], add=True)
        pltpu.async_copy(val[b3], acc_sfc.at[cidx[b3]], sems[b2], add=True)
        pltpu.async_copy(rq[b2], acc_sf2c.at[cidx[b3]], sems[b2], add=True)
        pltpu.async_copy(val[b3], acc_sfr.at[ridx[b3]], sems[b2], add=True)
        pltpu.async_copy(rq[b2], acc_sf2r.at[ridx[b3]], sems[b2], add=True)

    def wait_scatter(b3, b2):
        pltpu.make_async_copy(ones_v, acc_cnt.at[cidx[b3]], sems[b2]).wait()
        pltpu.make_async_copy(val[b3], acc_sfc.at[cidx[b3]], sems[b2]).wait()
        pltpu.make_async_copy(rq[b2], acc_sf2c.at[cidx[b3]], sems[b2]).wait()
        pltpu.make_async_copy(val[b3], acc_sfr.at[ridx[b3]], sems[b2]).wait()
        pltpu.make_async_copy(rq[b2], acc_sf2r.at[ridx[b3]], sems[b2]).wait()

    _db3_loop(NCH_S, issue, wait, process, wait_scatter)
    plsc.subcore_barrier()

    @pl.when(s == 0)
    def _():
        pltpu.sync_copy(acc_cnt, out_cnt.at[c])
        pltpu.sync_copy(acc_sfc, out_sfc.at[c])
        pltpu.sync_copy(acc_sf2c, out_sf2c.at[c])
        pltpu.sync_copy(acc_sfr, out_sfr.at[c])
        pltpu.sync_copy(acc_sf2r, out_sf2r.at[c])


CE_G = 2000
NCH_G = EPT // CE_G
NV_G = CE_G // L


@functools.partial(
    pl.kernel,
    out_type=jax.ShapeDtypeStruct((NC, NU), _f32),
    mesh=_MESH,
    compiler_params=_SC_PARAMS,
    scratch_types=[
        pltpu.VMEM((NI,), _f32),
        pltpu.VMEM((CE_G,), _i32),
        pltpu.VMEM((CE_G,), _i32),
        pltpu.VMEM((CE_G,), _i32),
        pltpu.VMEM((CE_G,), _i32),
        pltpu.VMEM((CE_G,), _i32),
        pltpu.VMEM((CE_G,), _i32),
        pltpu.VMEM((CE_G,), _f32),
        pltpu.VMEM((CE_G,), _f32),
        pltpu.VMEM((CE_G,), _f32),
        pltpu.VMEM_SHARED((NU,), _f32),
        pltpu.SemaphoreType.DMA,
        pltpu.SemaphoreType.DMA,
        pltpu.SemaphoreType.DMA,
        pltpu.SemaphoreType.DMA,
        pltpu.SemaphoreType.DMA,
    ],
)
def _pass_act(c1, r1, v1, g, zeros, out_act, g_v,
              cidx0, cidx1, cidx2, ridx0, ridx1, ridx2,
              val0, val1, val2, acc, seml0, seml1, seml2, sems0, sems1):
    c = lax.axis_index("c")
    s = lax.axis_index("s")
    base = _wid() * EPT
    cidx = (cidx0, cidx1, cidx2)
    ridx = (ridx0, ridx1, ridx2)
    val = (val0, val1, val2)
    seml = (seml0, seml1, seml2)
    sems = (sems0, sems1)

    @pl.when(s == 0)
    def _():
        pltpu.sync_copy(zeros, acc)

    pltpu.sync_copy(g, g_v)
    plsc.subcore_barrier()

    def issue(k, b3):
        e0 = base + k * CE_G
        pltpu.async_copy(c1.at[pl.ds(e0, CE_G)], cidx[b3], seml[b3])
        pltpu.async_copy(r1.at[pl.ds(e0, CE_G)], ridx[b3], seml[b3])
        pltpu.async_copy(v1.at[pl.ds(e0, CE_G)], val[b3], seml[b3])

    def wait(k, b3):
        e0 = base + k * CE_G
        pltpu.make_async_copy(c1.at[pl.ds(e0, CE_G)], cidx[b3],
                              seml[b3]).wait()
        pltpu.make_async_copy(r1.at[pl.ds(e0, CE_G)], ridx[b3],
                              seml[b3]).wait()
        pltpu.make_async_copy(v1.at[pl.ds(e0, CE_G)], val[b3],
                              seml[b3]).wait()

    def process(k, b3, b2):
        for r in range(NV_G):
            sl = pl.ds(r * L, L)
            gv = plsc.load_gather(g_v, [cidx[b3][sl]])
            val[b3][sl] = val[b3][sl] * gv
        pltpu.async_copy(val[b3], acc.at[ridx[b3]], sems[b2], add=True)

    def wait_scatter(b3, b2):
        pltpu.make_async_copy(val[b3], acc.at[ridx[b3]], sems[b2]).wait()

    _db3_loop(NCH_G, issue, wait, process, wait_scatter)
    plsc.subcore_barrier()

    @pl.when(s == 0)
    def _():
        pltpu.sync_copy(acc, out_act.at[c])


_HALF_U = NU // 2
_HALF_I = NI // 2


@functools.partial(
    pl.kernel,
    out_type=[
        jax.ShapeDtypeStruct((NE,), _f32),
        jax.ShapeDtypeStruct((NE,), _f32),
        jax.ShapeDtypeStruct((NE,), _f32),
        jax.ShapeDtypeStruct((NE,), _f32),
    ],
    mesh=_MESH,
    compiler_params=_SC_PARAMS,
    scratch_types=[
        pltpu.VMEM((_HALF_U,), _i32),
        pltpu.VMEM((_HALF_I,), _i32),
        pltpu.VMEM((CE_E,), _i32),
        pltpu.VMEM((CE_E,), _i32),
        pltpu.VMEM((CE_E,), _i32),
        pltpu.VMEM((CE_E,), _i32),
        pltpu.VMEM((CE_E,), _f32),
        pltpu.VMEM((CE_E,), _f32),
        pltpu.VMEM((CE_E,), _f32),
        pltpu.VMEM((CE_E,), _f32),
        pltpu.VMEM((CE_E,), _f32),
        pltpu.VMEM((CE_E,), _f32),
        pltpu.VMEM((CE_E,), _f32),
        pltpu.VMEM((CE_E,), _f32),
        pltpu.VMEM((CE_E,), _f32),
        pltpu.VMEM((CE_E,), _f32),
        pltpu.SemaphoreType.DMA,
        pltpu.SemaphoreType.DMA,
        pltpu.SemaphoreType.DMA,
        pltpu.SemaphoreType.DMA,
    ],
)
def _pass_out(r1, c1, v1, uwq, iwq, out1, out2, out3, out4,
              uwq_v, iwq_v, ridx0, ridx1, cidx0, cidx1, val0, val1,
              o1a, o1b, o2a, o2b, o3a, o3b, o4a, o4b,
              sem0, sem1, semw0, semw1):
    base = _wid() * EPT
    ridx = (ridx0, ridx1)
    cidx = (cidx0, cidx1)
    val = (val0, val1)
    ov = ((o1a, o2a, o3a, o4a), (o1b, o2b, o3b, o4b))
    outs = (out1, out2, out3, out4)
    sem = (sem0, sem1)
    semw = (semw0, semw1)

    pltpu.sync_copy(uwq, uwq_v)
    pltpu.sync_copy(iwq, iwq_v)

    def unpack(table, idx, half):
        lo = idx < half
        word = plsc.load_gather(table, [jnp.where(lo, idx, idx - half)])
        bits = jnp.where(lo, word << 16, word & jnp.int32(-65536))
        return plsc.bitcast(bits, _f32)

    def issue(k, b):
        e0 = base + k * CE_E
        pltpu.async_copy(r1.at[pl.ds(e0, CE_E)], ridx[b], sem[b])
        pltpu.async_copy(c1.at[pl.ds(e0, CE_E)], cidx[b], sem[b])
        pltpu.async_copy(v1.at[pl.ds(e0, CE_E)], val[b], sem[b])

    def wait(k, b):
        e0 = base + k * CE_E
        pltpu.make_async_copy(r1.at[pl.ds(e0, CE_E)], ridx[b], sem[b]).wait()
        pltpu.make_async_copy(c1.at[pl.ds(e0, CE_E)], cidx[b], sem[b]).wait()
        pltpu.make_async_copy(v1.at[pl.ds(e0, CE_E)], val[b], sem[b]).wait()

    def wait_writes(k, b):
        e0 = base + k * CE_E
        for q in range(4):
            pltpu.make_async_copy(
                ov[b][q], outs[q].at[pl.ds(e0, CE_E)], semw[b]).wait()

    def process(k, b):
        @pl.when(k >= 2)
        def _():
            wait_writes(k - 2, b)
        for r in range(NV_E):
            sl = pl.ds(r * L, L)
            uw = unpack(uwq_v, ridx[b][sl], _HALF_U)
            iw = unpack(iwq_v, cidx[b][sl], _HALF_I)
            v = val[b][sl]
            av = v * uw
            bv = v - av
            o1 = av * iw
            o4 = bv * iw
            ov[b][0][sl] = o1
            ov[b][1][sl] = bv - o4
            ov[b][2][sl] = av - o1
            ov[b][3][sl] = o4
        e0 = base + k * CE_E
        for q in range(4):
            pltpu.async_copy(ov[b][q], outs[q].at[pl.ds(e0, CE_E)], semw[b])

    issue(0, 0)

    def outer(ko, carry):
        for b in range(2):
            k = ko * 2 + b

            @pl.when(k + 1 < NCH_E)
            def _():
                issue(k + 1, 1 - b)

            wait(k, b)
            process(k, b)
        return carry

    lax.fori_loop(0, NCH_E // 2, outer, 0)
    wait_writes(NCH_E - 2, 0)
    wait_writes(NCH_E - 1, 1)


def _rn_bits(x):
    return lax.bitcast_convert_type(x, _i32) + jnp.int32(0x8000)


def _pack_halves(w):
    lo = w[0:4]
    hi = w[4:8]
    return (_rn_bits(hi) & jnp.int32(-65536)) | (
        (_rn_bits(lo) >> 16) & jnp.int32(0xFFFF))


def _det_w(low, high):
    al = jnp.maximum(low, 1e-6)
    ah = jnp.maximum(high, 1e-6)
    return al / (al + ah)


def _tc_item(cnt_p, sfc_p, sf2c_p):
    def body(cp, sp, qp, g_ref, iwq_ref):
        counts = cp[0:8] + cp[8:16]
        sf = sp[0:8] + sp[8:16]
        sf2 = qp[0:8] + qp[8:16]
        pop_raw = jnp.log1p(counts) * sf
        nrm = jnp.sqrt(jnp.sum(pop_raw * pop_raw))
        pop = pop_raw / (nrm + 1e-8)
        g_ref[...] = 1.0 / jnp.log1p(pop + 1e-8)
        p = 1.0 + pop
        il = p * sf * (1.0 / NU)
        ih = jnp.sqrt(p * p * sf2 + 1e-12)
        iwq_ref[...] = _pack_halves(_det_w(il, ih))

    return pl.pallas_call(
        body,
        out_shape=[
            jax.ShapeDtypeStruct((8, 12500), _f32),
            jax.ShapeDtypeStruct((4, 12500), _i32),
        ])(cnt_p, sfc_p, sf2c_p)


def _tc_user(act_p, sfr_p, sf2r_p):
    def body(ap, sp, qp, uwq_ref):
        ar = ap[0:8] + ap[8:16]
        nrm = jnp.sqrt(jnp.sum(ar * ar))
        a = 1.0 + ar / (nrm + 1e-8)
        sf = sp[0:8] + sp[8:16]
        sf2 = qp[0:8] + qp[8:16]
        ul = a * sf * (1.0 / NI)
        uh = jnp.sqrt(a * a * sf2 + 1e-12)
        uwq_ref[...] = _pack_halves(_det_w(ul, uh))

    return pl.pallas_call(
        body,
        out_shape=jax.ShapeDtypeStruct((4, 12500), _i32))(act_p, sfr_p,
                                                          sf2r_p)


def kernel(values, row_idx, col_idx):
    f = values.astype(_f32)
    zeros = jnp.zeros((NI,), _f32)
    ones1 = jnp.ones((CE_S,), _f32)

    cnt_p, sfc_p, sf2c_p, sfr_p, sf2r_p = _pass_stats(
        col_idx, row_idx, f, ones1, zeros)
    g, iwq = _tc_item(cnt_p.reshape(16, 12500), sfc_p.reshape(16, 12500),
                      sf2c_p.reshape(16, 12500))

    act_p = _pass_act(col_idx, row_idx, f, g.reshape(NI), zeros)
    uwq = _tc_user(act_p.reshape(16, 12500), sfr_p.reshape(16, 12500),
                   sf2r_p.reshape(16, 12500))

    o1, o2, o3, o4 = _pass_out(row_idx, col_idx, f, uwq.reshape(_HALF_U),
                               iwq.reshape(_HALF_I))
    return jnp.stack([o1, o2, o3, o4], axis=0)


# speedup vs baseline: 505.9939x; 1.1001x over previous
import functools

import jax
import jax.numpy as jnp
from jax import lax
from jax.experimental import pallas as pl
from jax.experimental.pallas import tpu as pltpu
from jax.experimental.pallas import tpu_sc as plsc

NU = 100000
NI = 100000
NE = 3200000
NC, NS, L = 2, 16, 16
NW = NC * NS
EPT = NE // NW

CE_A = 10000
NCH_A = EPT // CE_A
CE_B = 2000
NCH_B = EPT // CE_B
NV_B = CE_B // L
CE_C = 800
NCH_C = EPT // CE_C
NV_C = CE_C // L
CE_D = 2000
NCH_D = EPT // CE_D
NV_D = CE_D // L
CE_E = 2000
NCH_E = EPT // CE_E
NV_E = CE_E // L

_MESH = plsc.VectorSubcoreMesh(core_axis_name="c", subcore_axis_name="s")
_SC_PARAMS = pltpu.CompilerParams(needs_layout_passes=False,
                                  use_tc_tiling_on_sc=False)

_f32 = jnp.float32
_i32 = jnp.int32


def _wid():
    return lax.axis_index("s") * NC + lax.axis_index("c")


def _db3_loop(nchunk, issue_load, wait_load, process, wait_scatter):
    issue_load(0, 0)

    def traced_step(k, b3, b2):
        @pl.when(k + 1 < nchunk)
        def _():
            @pl.when(k >= 2)
            def _():
                wait_scatter((b3 + 1) % 3, b2)
            issue_load(k + 1, (b3 + 1) % 3)
        wait_load(k, b3)
        process(k, b3, b2)

    nfull = nchunk // 6

    def outer(ko, carry):
        k0 = ko * 6
        for b in range(6):
            traced_step(k0 + b, b % 3, b % 2)
        return carry

    lax.fori_loop(0, nfull, outer, 0)
    for k in range(nfull * 6, nchunk):
        if k + 1 < nchunk:
            if k >= 2:
                wait_scatter((k + 1) % 3, k % 2)
            issue_load(k + 1, (k + 1) % 3)
        wait_load(k, k % 3)
        process(k, k % 3, k % 2)
    for j in range(max(0, nchunk - 3), nchunk):
        wait_scatter(j % 3, j % 2)


CE_S = 4000
NCH_S = EPT // CE_S
NV_S = CE_S // L


@functools.partial(
    pl.kernel,
    out_type=[
        jax.ShapeDtypeStruct((NC, NI), _f32),
        jax.ShapeDtypeStruct((NC, NI), _f32),
        jax.ShapeDtypeStruct((NC, NI), _f32),
        jax.ShapeDtypeStruct((NC, NU), _f32),
        jax.ShapeDtypeStruct((NC, NU), _f32),
    ],
    mesh=_MESH,
    compiler_params=_SC_PARAMS,
    scratch_types=[
        pltpu.VMEM((CE_S,), _i32),
        pltpu.VMEM((CE_S,), _i32),
        pltpu.VMEM((CE_S,), _i32),
        pltpu.VMEM((CE_S,), _i32),
        pltpu.VMEM((CE_S,), _i32),
        pltpu.VMEM((CE_S,), _i32),
        pltpu.VMEM((CE_S,), _f32),
        pltpu.VMEM((CE_S,), _f32),
        pltpu.VMEM((CE_S,), _f32),
        pltpu.VMEM((CE_S,), _f32),
        pltpu.VMEM((CE_S,), _f32),
        pltpu.VMEM((CE_S,), _f32),
        pltpu.VMEM_SHARED((NI,), _f32),
        pltpu.VMEM_SHARED((NI,), _f32),
        pltpu.VMEM_SHARED((NI,), _f32),
        pltpu.VMEM_SHARED((NU,), _f32),
        pltpu.VMEM_SHARED((NU,), _f32),
        pltpu.SemaphoreType.DMA,
        pltpu.SemaphoreType.DMA,
        pltpu.SemaphoreType.DMA,
        pltpu.SemaphoreType.DMA,
        pltpu.SemaphoreType.DMA,
    ],
)
def _pass_stats(c1, r1, v1, ones1, zeros,
                out_cnt, out_sfc, out_sf2c, out_sfr, out_sf2r,
                cidx0, cidx1, cidx2, ridx0, ridx1, ridx2,
                val0, val1, val2, ones_v, rq0, rq1,
                acc_cnt, acc_sfc, acc_sf2c, acc_sfr, acc_sf2r,
                seml0, seml1, seml2, sems0, sems1):
    c = lax.axis_index("c")
    s = lax.axis_index("s")
    base = _wid() * EPT
    cidx = (cidx0, cidx1, cidx2)
    ridx = (ridx0, ridx1, ridx2)
    val = (val0, val1, val2)
    rq = (rq0, rq1)
    seml = (seml0, seml1, seml2)
    sems = (sems0, sems1)

    @pl.when(s == 0)
    def _():
        pltpu.sync_copy(zeros, acc_cnt)
        pltpu.sync_copy(zeros, acc_sfc)
        pltpu.sync_copy(zeros, acc_sf2c)
        pltpu.sync_copy(zeros.at[pl.ds(0, NU)], acc_sfr)
        pltpu.sync_copy(zeros.at[pl.ds(0, NU)], acc_sf2r)

    pltpu.sync_copy(ones1, ones_v)
    plsc.subcore_barrier()

    def issue(k, b3):
        e0 = base + k * CE_S
        pltpu.async_copy(c1.at[pl.ds(e0, CE_S)], cidx[b3], seml[b3])
        pltpu.async_copy(r1.at[pl.ds(e0, CE_S)], ridx[b3], seml[b3])
        pltpu.async_copy(v1.at[pl.ds(e0, CE_S)], val[b3], seml[b3])

    def wait(k, b3):
        e0 = base + k * CE_S
        pltpu.make_async_copy(c1.at[pl.ds(e0, CE_S)], cidx[b3],
                              seml[b3]).wait()
        pltpu.make_async_copy(r1.at[pl.ds(e0, CE_S)], ridx[b3],
                              seml[b3]).wait()
        pltpu.make_async_copy(v1.at[pl.ds(e0, CE_S)], val[b3],
                              seml[b3]).wait()

    def process(k, b3, b2):
        for r in range(NV_S):
            sl = pl.ds(r * L, L)
            v = val[b3][sl]
            rq[b2][sl] = v * v
        pltpu.async_copy(ones_v, acc_cnt.at[cidx[b3]], sems[b2], add=True)
        pltpu.async_copy(val[b3], acc_sfc.at[cidx[b3]], sems[b2], add=True)
        pltpu.async_copy(rq[b2], acc_sf2c.at[cidx[b3]], sems[b2], add=True)
        pltpu.async_copy(val[b3], acc_sfr.at[ridx[b3]], sems[b2], add=True)
        pltpu.async_copy(rq[b2], acc_sf2r.at[ridx[b3]], sems[b2], add=True)

    def wait_scatter(b3, b2):
        pltpu.make_async_copy(ones_v, acc_cnt.at[cidx[b3]], sems[b2]).wait()
        pltpu.make_async_copy(val[b3], acc_sfc.at[cidx[b3]], sems[b2]).wait()
        pltpu.make_async_copy(rq[b2], acc_sf2c.at[cidx[b3]], sems[b2]).wait()
        pltpu.make_async_copy(val[b3], acc_sfr.at[ridx[b3]], sems[b2]).wait()
        pltpu.make_async_copy(rq[b2], acc_sf2r.at[ridx[b3]], sems[b2]).wait()

    _db3_loop(NCH_S, issue, wait, process, wait_scatter)
    plsc.subcore_barrier()

    @pl.when(s == 0)
    def _():
        pltpu.sync_copy(acc_cnt, out_cnt.at[c])
        pltpu.sync_copy(acc_sfc, out_sfc.at[c])
        pltpu.sync_copy(acc_sf2c, out_sf2c.at[c])
        pltpu.sync_copy(acc_sfr, out_sfr.at[c])
        pltpu.sync_copy(acc_sf2r, out_sf2r.at[c])


CE_G = 2000
NCH_G = EPT // CE_G
NV_G = CE_G // L


@functools.partial(
    pl.kernel,
    out_type=jax.ShapeDtypeStruct((NC, NU), _f32),
    mesh=_MESH,
    compiler_params=_SC_PARAMS,
    scratch_types=[
        pltpu.VMEM((NI,), _f32),
        pltpu.VMEM((CE_G,), _i32),
        pltpu.VMEM((CE_G,), _i32),
        pltpu.VMEM((CE_G,), _i32),
        pltpu.VMEM((CE_G,), _i32),
        pltpu.VMEM((CE_G,), _i32),
        pltpu.VMEM((CE_G,), _i32),
        pltpu.VMEM((CE_G,), _f32),
        pltpu.VMEM((CE_G,), _f32),
        pltpu.VMEM((CE_G,), _f32),
        pltpu.VMEM_SHARED((NU,), _f32),
        pltpu.SemaphoreType.DMA,
        pltpu.SemaphoreType.DMA,
        pltpu.SemaphoreType.DMA,
        pltpu.SemaphoreType.DMA,
        pltpu.SemaphoreType.DMA,
    ],
)
def _pass_act(c1, r1, v1, g, zeros, out_act, g_v,
              cidx0, cidx1, cidx2, ridx0, ridx1, ridx2,
              val0, val1, val2, acc, seml0, seml1, seml2, sems0, sems1):
    c = lax.axis_index("c")
    s = lax.axis_index("s")
    base = _wid() * EPT
    cidx = (cidx0, cidx1, cidx2)
    ridx = (ridx0, ridx1, ridx2)
    val = (val0, val1, val2)
    seml = (seml0, seml1, seml2)
    sems = (sems0, sems1)

    @pl.when(s == 0)
    def _():
        pltpu.sync_copy(zeros, acc)

    pltpu.sync_copy(g, g_v)
    plsc.subcore_barrier()

    def issue(k, b3):
        e0 = base + k * CE_G
        pltpu.async_copy(c1.at[pl.ds(e0, CE_G)], cidx[b3], seml[b3])
        pltpu.async_copy(r1.at[pl.ds(e0, CE_G)], ridx[b3], seml[b3])
        pltpu.async_copy(v1.at[pl.ds(e0, CE_G)], val[b3], seml[b3])

    def wait(k, b3):
        e0 = base + k * CE_G
        pltpu.make_async_copy(c1.at[pl.ds(e0, CE_G)], cidx[b3],
                              seml[b3]).wait()
        pltpu.make_async_copy(r1.at[pl.ds(e0, CE_G)], ridx[b3],
                              seml[b3]).wait()
        pltpu.make_async_copy(v1.at[pl.ds(e0, CE_G)], val[b3],
                              seml[b3]).wait()

    def process(k, b3, b2):
        for r in range(NV_G):
            sl = pl.ds(r * L, L)
            gv = plsc.load_gather(g_v, [cidx[b3][sl]])
            val[b3][sl] = val[b3][sl] * gv
        pltpu.async_copy(val[b3], acc.at[ridx[b3]], sems[b2], add=True)

    def wait_scatter(b3, b2):
        pltpu.make_async_copy(val[b3], acc.at[ridx[b3]], sems[b2]).wait()

    _db3_loop(NCH_G, issue, wait, process, wait_scatter)
    plsc.subcore_barrier()

    @pl.when(s == 0)
    def _():
        pltpu.sync_copy(acc, out_act.at[c])


_HALF_U = NU // 2
_HALF_I = NI // 2
CE_E = 2000
NCH_E = EPT // CE_E
NV_E = CE_E // L


@functools.partial(
    pl.kernel,
    out_type=[
        jax.ShapeDtypeStruct((NE,), _f32),
        jax.ShapeDtypeStruct((NE,), _f32),
    ],
    mesh=_MESH,
    compiler_params=_SC_PARAMS,
    scratch_types=[
        pltpu.VMEM((_HALF_U,), _i32),
        pltpu.VMEM((_HALF_I,), _i32),
        pltpu.VMEM((CE_E,), _i32),
        pltpu.VMEM((CE_E,), _i32),
        pltpu.VMEM((CE_E,), _i32),
        pltpu.VMEM((CE_E,), _i32),
        pltpu.VMEM((CE_E,), _f32),
        pltpu.VMEM((CE_E,), _f32),
        pltpu.VMEM((CE_E,), _f32),
        pltpu.VMEM((CE_E,), _f32),
        pltpu.SemaphoreType.DMA,
        pltpu.SemaphoreType.DMA,
        pltpu.SemaphoreType.DMA,
        pltpu.SemaphoreType.DMA,
    ],
)
def _pass_gw(r1, c1, uwq, iwq, outu, outi,
             uwq_v, iwq_v, ridx0, ridx1, cidx0, cidx1,
             ou0, ou1, oi0, oi1, sem0, sem1, semw0, semw1):
    base = _wid() * EPT
    ridx = (ridx0, ridx1)
    cidx = (cidx0, cidx1)
    ou = (ou0, ou1)
    oi = (oi0, oi1)
    sem = (sem0, sem1)
    semw = (semw0, semw1)

    pltpu.sync_copy(uwq, uwq_v)
    pltpu.sync_copy(iwq, iwq_v)

    def unpack(table, idx, half):
        lo = idx < half
        word = plsc.load_gather(table, [jnp.where(lo, idx, idx - half)])
        bits = jnp.where(lo, word << 16, word & jnp.int32(-65536))
        return plsc.bitcast(bits, _f32)

    def issue(k, b):
        e0 = base + k * CE_E
        pltpu.async_copy(r1.at[pl.ds(e0, CE_E)], ridx[b], sem[b])
        pltpu.async_copy(c1.at[pl.ds(e0, CE_E)], cidx[b], sem[b])

    def wait(k, b):
        e0 = base + k * CE_E
        pltpu.make_async_copy(r1.at[pl.ds(e0, CE_E)], ridx[b], sem[b]).wait()
        pltpu.make_async_copy(c1.at[pl.ds(e0, CE_E)], cidx[b], sem[b]).wait()

    def wait_writes(k, b):
        e0 = base + k * CE_E
        pltpu.make_async_copy(ou[b], outu.at[pl.ds(e0, CE_E)],
                              semw[b]).wait()
        pltpu.make_async_copy(oi[b], outi.at[pl.ds(e0, CE_E)],
                              semw[b]).wait()

    def process(k, b):
        @pl.when(k >= 2)
        def _():
            wait_writes(k - 2, b)
        for r in range(NV_E):
            sl = pl.ds(r * L, L)
            ou[b][sl] = unpack(uwq_v, ridx[b][sl], _HALF_U)
            oi[b][sl] = unpack(iwq_v, cidx[b][sl], _HALF_I)
        e0 = base + k * CE_E
        pltpu.async_copy(ou[b], outu.at[pl.ds(e0, CE_E)], semw[b])
        pltpu.async_copy(oi[b], outi.at[pl.ds(e0, CE_E)], semw[b])

    issue(0, 0)

    def outer(ko, carry):
        for b in range(2):
            k = ko * 2 + b

            @pl.when(k + 1 < NCH_E)
            def _():
                issue(k + 1, 1 - b)

            wait(k, b)
            process(k, b)
        return carry

    lax.fori_loop(0, NCH_E // 2, outer, 0)
    wait_writes(NCH_E - 2, 0)
    wait_writes(NCH_E - 1, 1)


_CB = 128000
_NB = NE // _CB


def _tc_quadrants(f, uwg, iwg):
    def body(f_ref, u_ref, i_ref, out_ref):
        v = f_ref[...]
        uw = u_ref[...]
        iw = i_ref[...]
        av = v * uw
        bv = v - av
        o1 = av * iw
        o4 = bv * iw
        out_ref[0, :] = o1
        out_ref[1, :] = bv - o4
        out_ref[2, :] = av - o1
        out_ref[3, :] = o4

    in_spec = pl.BlockSpec((_CB,), lambda i: (i,))
    return pl.pallas_call(
        body,
        grid=(_NB,),
        in_specs=[in_spec, in_spec, in_spec],
        out_specs=pl.BlockSpec((4, _CB), lambda i: (0, i)),
        out_shape=jax.ShapeDtypeStruct((4, NE), _f32),
    )(f, uwg, iwg)


def _rn_bits(x):
    return lax.bitcast_convert_type(x, _i32) + jnp.int32(0x8000)


def _pack_halves(w):
    lo = w[0:4]
    hi = w[4:8]
    return (_rn_bits(hi) & jnp.int32(-65536)) | (
        (_rn_bits(lo) >> 16) & jnp.int32(0xFFFF))


def _det_w(low, high):
    al = jnp.maximum(low, 1e-6)
    ah = jnp.maximum(high, 1e-6)
    return al / (al + ah)


def _tc_item(cnt_p, sfc_p, sf2c_p):
    def body(cp, sp, qp, g_ref, iwq_ref):
        counts = cp[0:8] + cp[8:16]
        sf = sp[0:8] + sp[8:16]
        sf2 = qp[0:8] + qp[8:16]
        pop_raw = jnp.log1p(counts) * sf
        nrm = jnp.sqrt(jnp.sum(pop_raw * pop_raw))
        pop = pop_raw / (nrm + 1e-8)
        g_ref[...] = 1.0 / jnp.log1p(pop + 1e-8)
        p = 1.0 + pop
        il = p * sf * (1.0 / NU)
        ih = jnp.sqrt(p * p * sf2 + 1e-12)
        iwq_ref[...] = _pack_halves(_det_w(il, ih))

    return pl.pallas_call(
        body,
        out_shape=[
            jax.ShapeDtypeStruct((8, 12500), _f32),
            jax.ShapeDtypeStruct((4, 12500), _i32),
        ])(cnt_p, sfc_p, sf2c_p)


def _tc_user(act_p, sfr_p, sf2r_p):
    def body(ap, sp, qp, uwq_ref):
        ar = ap[0:8] + ap[8:16]
        nrm = jnp.sqrt(jnp.sum(ar * ar))
        a = 1.0 + ar / (nrm + 1e-8)
        sf = sp[0:8] + sp[8:16]
        sf2 = qp[0:8] + qp[8:16]
        ul = a * sf * (1.0 / NI)
        uh = jnp.sqrt(a * a * sf2 + 1e-12)
        uwq_ref[...] = _pack_halves(_det_w(ul, uh))

    return pl.pallas_call(
        body,
        out_shape=jax.ShapeDtypeStruct((4, 12500), _i32))(act_p, sfr_p,
                                                          sf2r_p)


def kernel(values, row_idx, col_idx):
    f = values.astype(_f32)
    zeros = jnp.zeros((NI,), _f32)
    ones1 = jnp.ones((CE_S,), _f32)

    cnt_p, sfc_p, sf2c_p, sfr_p, sf2r_p = _pass_stats(
        col_idx, row_idx, f, ones1, zeros)
    g, iwq = _tc_item(cnt_p.reshape(16, 12500), sfc_p.reshape(16, 12500),
                      sf2c_p.reshape(16, 12500))

    act_p = _pass_act(col_idx, row_idx, f, g.reshape(NI), zeros)
    uwq = _tc_user(act_p.reshape(16, 12500), sfr_p.reshape(16, 12500),
                   sf2r_p.reshape(16, 12500))

    uwg, iwg = _pass_gw(row_idx, col_idx, uwq.reshape(_HALF_U),
                        iwq.reshape(_HALF_I))
    return _tc_quadrants(f, uwg, iwg)
